# Initial kernel scaffold; baseline (speedup 1.0000x reference)
import jax
import jax.numpy as jnp
from jax.experimental import pallas as pl


def kernel(x, edge_index, i, W1, b1, W2, b2, W3, b3, W4, b4, W5, b5, Wd, bd):
    raise NotImplementedError("write your pallas kernel here")


# speedup vs baseline: 4.3896x; 4.3896x over previous
import functools

import jax
import jax.numpy as jnp
from jax import lax
from jax.experimental import pallas as pl
from jax.experimental.pallas import tpu as pltpu
from jax.experimental.pallas import tpu_sc as plsc

N = 10000
E = 320000
G = 64
N_PAD = 10240
E_PAD = 327680
NC = 2
NS = 16
CH = 128
EPT = E_PAD // (NC * NS)
NCHUNK = EPT // CH
RPT = N_PAD // NS
RBLK = 1024
NB = N_PAD // RBLK


@functools.lru_cache(maxsize=None)
def _make_sc_agg(d):
    mesh = plsc.VectorSubcoreMesh(
        core_axis_name="c", subcore_axis_name="s",
        num_cores=NC, num_subcores=NS)

    @functools.partial(
        pl.kernel,
        mesh=mesh,
        compiler_params=pltpu.CompilerParams(use_tc_tiling_on_sc=False),
        out_type=jax.ShapeDtypeStruct((NC * N_PAD, d), jnp.float32),
        scratch_types=[
            pltpu.VMEM((CH,), jnp.int32),
            pltpu.VMEM((CH,), jnp.int32),
            pltpu.VMEM((CH, d), jnp.float32),
            pltpu.VMEM_SHARED((N_PAD, d), jnp.float32),
            pltpu.SemaphoreType.DMA,
        ],
    )
    def agg(vals, src, dst, zeros, out, src_v, dst_v, rows_v, acc, sem):
        c = lax.axis_index("c")
        s = lax.axis_index("s")
        pltpu.sync_copy(zeros, acc.at[pl.ds(s * RPT, RPT)])
        plsc.subcore_barrier()
        base = (c * NS + s) * EPT

        def body(g, carry):
            off = base + g * CH
            pltpu.sync_copy(src.at[pl.ds(off, CH)], src_v)
            pltpu.sync_copy(dst.at[pl.ds(off, CH)], dst_v)
            pltpu.async_copy(vals.at[src_v], rows_v, sem).wait()
            pltpu.sync_copy(rows_v, acc.at[dst_v], add=True)
            return carry

        lax.fori_loop(0, NCHUNK, body, 0)
        plsc.subcore_barrier()
        pltpu.sync_copy(
            acc.at[pl.ds(s * RPT, RPT)],
            out.at[pl.ds(c * N_PAD + s * RPT, RPT)],
        )

    return agg


def _sc_aggregate(vals, src, dst):
    d = vals.shape[1]
    zeros = jnp.zeros((RPT, d), jnp.float32)
    return _make_sc_agg(d)(vals, src, dst, zeros)


def _mm(a, b):
    return lax.dot_general(
        a, b, (((1,), (0,)), ((), ())),
        precision=lax.Precision.HIGHEST,
        preferred_element_type=jnp.float32,
    )


def _l2relu(t):
    nrm = lax.rsqrt(jnp.maximum(jnp.sum(t * t, -1, keepdims=True), 1e-12))
    return jnp.maximum(t * nrm, 0.0)


def _row_spec(d):
    return pl.BlockSpec((RBLK, d), lambda k: (k, 0))


def _part_specs(d):
    return (
        pl.BlockSpec((RBLK, d), lambda k: (k, 0)),
        pl.BlockSpec((RBLK, d), lambda k: (k + NB, 0)),
    )


def _full_spec(shape):
    nd = len(shape)
    return pl.BlockSpec(shape, lambda k: (0,) * nd)


def _tc1_body(x_ref, w_ref, b_ref, s1_ref, pd1_ref):
    x = x_ref[...]
    w = w_ref[...]
    s1_ref[...] = _mm(x, w[:128]) + b_ref[0:1, :]
    p = _mm(x, w[128:])
    onescol = (lax.broadcasted_iota(jnp.int32, (RBLK, 16), 1) == 0).astype(jnp.float32)
    pd1_ref[...] = jnp.concatenate([p, onescol], axis=1)


def _tc1(x, w1, b1):
    return pl.pallas_call(
        _tc1_body,
        grid=(NB,),
        in_specs=[_row_spec(128), _full_spec((256, 64)), _full_spec((8, 64))],
        out_specs=[_row_spec(64), _row_spec(80)],
        out_shape=[
            jax.ShapeDtypeStruct((N_PAD, 64), jnp.float32),
            jax.ShapeDtypeStruct((N_PAD, 80), jnp.float32),
        ],
    )(x, w1, b1)


def _tc2_body(s1_ref, pa_ref, pb_ref, h1_ref, dinv_ref):
    p = pa_ref[...] + pb_ref[...]
    deg = jnp.sum(p[:, 64:80], -1, keepdims=True)
    dinv = 1.0 / jnp.maximum(deg, 1.0)
    t = s1_ref[...] + p[:, :64] * dinv
    h1_ref[...] = _l2relu(t)
    dinv_ref[...] = jnp.broadcast_to(dinv, (RBLK, 64))


def _tc2(s1, part1):
    pa, pb = _part_specs(80)
    return pl.pallas_call(
        _tc2_body,
        grid=(NB,),
        in_specs=[_row_spec(64), pa, pb],
        out_specs=[_row_spec(64), _row_spec(64)],
        out_shape=[
            jax.ShapeDtypeStruct((N_PAD, 64), jnp.float32),
            jax.ShapeDtypeStruct((N_PAD, 64), jnp.float32),
        ],
    )(s1, part1, part1)


def _tc3(h1, part2, dinv, w2, b2, w3, b3):
    pa, pb = _part_specs(64)

    def body(h1_ref, pa_ref, pb_ref, dinv_ref, w2_ref, b2_ref, w3_ref, b3_ref,
             s3_ref, p3_ref):
        h1 = h1_ref[...]
        agg2 = (pa_ref[...] + pb_ref[...]) * dinv_ref[...]
        w2 = w2_ref[...]
        h2 = _l2relu(_mm(h1, w2[:64]) + _mm(agg2, w2[64:]) + b2_ref[0:1, :])
        w3 = w3_ref[...]
        s3_ref[...] = _mm(h2, w3[:256]) + b3_ref[0:1, :]
        p3_ref[...] = _mm(h2, w3[256:])

    return pl.pallas_call(
        body,
        grid=(NB,),
        in_specs=[_row_spec(64), pa, pb, _row_spec(64),
                  _full_spec((128, 256)), _full_spec((8, 256)),
                  _full_spec((512, 32)), _full_spec((8, 32))],
        out_specs=[_row_spec(32), _row_spec(32)],
        out_shape=[
            jax.ShapeDtypeStruct((N_PAD, 32), jnp.float32),
            jax.ShapeDtypeStruct((N_PAD, 32), jnp.float32),
        ],
    )(h1, part2, part2, dinv, w2, b2, w3, b3)


def _tc4(s3, part3, dinv):
    pa, pb = _part_specs(32)

    def body(s3_ref, pa_ref, pb_ref, dinv_ref, h3_ref):
        agg3 = (pa_ref[...] + pb_ref[...]) * dinv_ref[:, :32]
        h3_ref[...] = _l2relu(s3_ref[...] + agg3)

    return pl.pallas_call(
        body,
        grid=(NB,),
        in_specs=[_row_spec(32), pa, pb, _row_spec(64)],
        out_specs=_row_spec(32),
        out_shape=jax.ShapeDtypeStruct((N_PAD, 32), jnp.float32),
    )(s3, part3, part3, dinv)


def _tc5(h3, part4, dinv, w4, b4, w5, b5):
    pa, pb = _part_specs(32)

    def body(h3_ref, pa_ref, pb_ref, dinv_ref, w4_ref, b4_ref, w5_ref, b5_ref,
             s5_ref, p5_ref):
        h3 = h3_ref[...]
        agg4 = (pa_ref[...] + pb_ref[...]) * dinv_ref[:, :32]
        w4 = w4_ref[...]
        h4 = _l2relu(_mm(h3, w4[:32]) + _mm(agg4, w4[32:]) + b4_ref[0:1, :])
        w5 = w5_ref[...]
        s5_ref[...] = _mm(h4, w5[:64]) + b5_ref[0:1, :]
        p5_ref[...] = _mm(h4, w5[64:])

    return pl.pallas_call(
        body,
        grid=(NB,),
        in_specs=[_row_spec(32), pa, pb, _row_spec(64),
                  _full_spec((64, 64)), _full_spec((8, 64)),
                  _full_spec((128, 64)), _full_spec((8, 64))],
        out_specs=[_row_spec(64), _row_spec(64)],
        out_shape=[
            jax.ShapeDtypeStruct((N_PAD, 64), jnp.float32),
            jax.ShapeDtypeStruct((N_PAD, 64), jnp.float32),
        ],
    )(h3, part4, part4, dinv, w4, b4, w5, b5)


def _tc6(s5, part5, dinv, ib, wd, bd):
    pa, pb = _part_specs(64)

    def body(s5_ref, pa_ref, pb_ref, dinv_ref, ib_ref, wd_ref, bd_ref,
             out_ref, psum, cnt):
        k = pl.program_id(0)

        @pl.when(k == 0)
        def _():
            psum[...] = jnp.zeros((G, 64), jnp.float32)
            cnt[...] = jnp.zeros((G, 16), jnp.float32)

        agg5 = (pa_ref[...] + pb_ref[...]) * dinv_ref[...]
        h5 = _l2relu(s5_ref[...] + agg5)
        m = (ib_ref[...] == lax.broadcasted_iota(jnp.int32, (RBLK, G), 1))
        m = m.astype(jnp.float32)
        cT = (((0,), (0,)), ((), ()))
        psum[...] += lax.dot_general(
            m, h5, cT, precision=lax.Precision.HIGHEST,
            preferred_element_type=jnp.float32)
        cnt[...] += lax.dot_general(
            m, jnp.ones((RBLK, 16), jnp.float32), cT,
            precision=lax.Precision.HIGHEST,
            preferred_element_type=jnp.float32)

        @pl.when(k == NB - 1)
        def _():
            pooled = psum[...] * (1.0 / jnp.maximum(cnt[:, 0:1], 1.0))
            out_ref[...] = jnp.tanh(_mm(pooled, wd_ref[...]) + bd_ref[0:1, :])

    return pl.pallas_call(
        body,
        grid=(NB,),
        in_specs=[_row_spec(64), pa, pb, _row_spec(64), _row_spec(64),
                  _full_spec((64, 16)), _full_spec((8, 16))],
        out_specs=_full_spec((G, 16)),
        out_shape=jax.ShapeDtypeStruct((G, 16), jnp.float32),
        scratch_shapes=[
            pltpu.VMEM((G, 64), jnp.float32),
            pltpu.VMEM((G, 16), jnp.float32),
        ],
    )(s5, part5, part5, dinv, ib, wd, bd)


def _pad_bias(b, d):
    return jnp.broadcast_to(b[None, :], (8, d))


def kernel(x, edge_index, i, W1, b1, W2, b2, W3, b3, W4, b4, W5, b5, Wd, bd):
    f32 = jnp.float32
    xp = jnp.pad(x, ((0, N_PAD - N), (0, 0)))
    src = jnp.concatenate([edge_index[0], jnp.zeros((E_PAD - E,), jnp.int32)])
    dst = jnp.concatenate([edge_index[1], jnp.full((E_PAD - E,), N, jnp.int32)])
    ip = jnp.concatenate([i, jnp.full((N_PAD - N,), G, jnp.int32)])
    ib = jnp.broadcast_to(ip[:, None], (N_PAD, G))
    wd16 = jnp.pad(Wd, ((0, 0), (0, 6)))
    bd16 = _pad_bias(jnp.pad(bd, (0, 6)), 16)

    s1, pd1 = _tc1(xp, W1, _pad_bias(b1, 64))
    part1 = _sc_aggregate(pd1, src, dst)
    h1, dinv = _tc2(s1, part1)

    part2 = _sc_aggregate(h1, src, dst)
    s3, p3 = _tc3(h1, part2, dinv, W2, _pad_bias(b2, 256), W3, _pad_bias(b3, 32))

    part3 = _sc_aggregate(p3, src, dst)
    h3 = _tc4(s3, part3, dinv)

    part4 = _sc_aggregate(h3, src, dst)
    s5, p5 = _tc5(h3, part4, dinv, W4, _pad_bias(b4, 64), W5, _pad_bias(b5, 64))

    part5 = _sc_aggregate(p5, src, dst)
    out16 = _tc6(s5, part5, dinv, ib, wd16, bd16)
    return out16[:, :10].astype(f32)


# speedup vs baseline: 5.7805x; 1.3169x over previous
import functools

import jax
import jax.numpy as jnp
from jax import lax
from jax.experimental import pallas as pl
from jax.experimental.pallas import tpu as pltpu
from jax.experimental.pallas import tpu_sc as plsc

N = 10000
E = 320000
G = 64
N_PAD = 10240
E_PAD = 327680
NC = 2
NS = 16
CH = 128
EPT = E_PAD // (NC * NS)
NCHUNK = EPT // CH
RPT = N_PAD // NS
RBLK = 1024
NB = N_PAD // RBLK


_K_FOR_D = {80: 5, 64: 8, 32: 16}


@functools.lru_cache(maxsize=None)
def _make_sc_agg(d):
    K = _K_FOR_D[d]
    NGRP = NCHUNK // K
    mesh = plsc.VectorSubcoreMesh(
        core_axis_name="c", subcore_axis_name="s",
        num_cores=NC, num_subcores=NS)

    @functools.partial(
        pl.kernel,
        mesh=mesh,
        compiler_params=pltpu.CompilerParams(use_tc_tiling_on_sc=False),
        out_type=jax.ShapeDtypeStruct((NC * N_PAD, d), jnp.float32),
        scratch_types=(
            [pltpu.VMEM((NCHUNK, CH), jnp.int32)]
            + [pltpu.VMEM((NCHUNK, CH), jnp.int32)]
            + [pltpu.VMEM((CH, d), jnp.float32) for _ in range(K)]
            + [pltpu.VMEM_SHARED((N_PAD, d), jnp.float32)]
            + [pltpu.SemaphoreType.DMA]
        ),
    )
    def agg(vals, src2, dst2, zeros, out, *scr):
        src_all = scr[0]
        dst_all = scr[1]
        rows = scr[2:2 + K]
        acc = scr[2 + K]
        gsem = scr[3 + K]
        c = lax.axis_index("c")
        s = lax.axis_index("s")
        pltpu.sync_copy(zeros, acc.at[pl.ds(s * RPT, RPT)])
        row0 = (c * NS + s) * NCHUNK
        pltpu.sync_copy(src2.at[pl.ds(row0, NCHUNK)], src_all)
        pltpu.sync_copy(dst2.at[pl.ds(row0, NCHUNK)], dst_all)
        plsc.subcore_barrier()

        def group(j, carry):
            g0 = j * K
            for b in range(K):
                pltpu.async_copy(vals.at[src_all.at[g0 + b]], rows[b], gsem)
            for b in range(K):
                pltpu.make_async_copy(
                    vals.at[src_all.at[g0 + b]], rows[b], gsem).wait()
            for b in range(K):
                pltpu.sync_copy(rows[b], acc.at[dst_all.at[g0 + b]], add=True)
            return carry

        lax.fori_loop(0, NGRP, group, 0)
        plsc.subcore_barrier()
        pltpu.sync_copy(
            acc.at[pl.ds(s * RPT, RPT)],
            out.at[pl.ds(c * N_PAD + s * RPT, RPT)],
        )

    return agg


def _sc_aggregate(vals, src, dst):
    d = vals.shape[1]
    zeros = jnp.zeros((RPT, d), jnp.float32)
    return _make_sc_agg(d)(vals, src, dst, zeros)


def _mm(a, b):
    return lax.dot_general(
        a, b, (((1,), (0,)), ((), ())),
        precision=lax.Precision.HIGHEST,
        preferred_element_type=jnp.float32,
    )


def _l2relu(t):
    nrm = lax.rsqrt(jnp.maximum(jnp.sum(t * t, -1, keepdims=True), 1e-12))
    return jnp.maximum(t * nrm, 0.0)


def _row_spec(d):
    return pl.BlockSpec((RBLK, d), lambda k: (k, 0))


def _part_specs(d):
    return (
        pl.BlockSpec((RBLK, d), lambda k: (k, 0)),
        pl.BlockSpec((RBLK, d), lambda k: (k + NB, 0)),
    )


def _full_spec(shape):
    nd = len(shape)
    return pl.BlockSpec(shape, lambda k: (0,) * nd)


def _tc1_body(x_ref, w_ref, b_ref, s1_ref, pd1_ref):
    x = x_ref[...]
    w = w_ref[...]
    s1_ref[...] = _mm(x, w[:128]) + b_ref[0:1, :]
    p = _mm(x, w[128:])
    onescol = (lax.broadcasted_iota(jnp.int32, (RBLK, 16), 1) == 0).astype(jnp.float32)
    pd1_ref[...] = jnp.concatenate([p, onescol], axis=1)


def _tc1(x, w1, b1):
    return pl.pallas_call(
        _tc1_body,
        grid=(NB,),
        in_specs=[_row_spec(128), _full_spec((256, 64)), _full_spec((8, 64))],
        out_specs=[_row_spec(64), _row_spec(80)],
        out_shape=[
            jax.ShapeDtypeStruct((N_PAD, 64), jnp.float32),
            jax.ShapeDtypeStruct((N_PAD, 80), jnp.float32),
        ],
    )(x, w1, b1)


def _tc2_body(s1_ref, pa_ref, pb_ref, h1_ref, dinv_ref):
    p = pa_ref[...] + pb_ref[...]
    deg = jnp.sum(p[:, 64:80], -1, keepdims=True)
    dinv = 1.0 / jnp.maximum(deg, 1.0)
    t = s1_ref[...] + p[:, :64] * dinv
    h1_ref[...] = _l2relu(t)
    dinv_ref[...] = jnp.broadcast_to(dinv, (RBLK, 64))


def _tc2(s1, part1):
    pa, pb = _part_specs(80)
    return pl.pallas_call(
        _tc2_body,
        grid=(NB,),
        in_specs=[_row_spec(64), pa, pb],
        out_specs=[_row_spec(64), _row_spec(64)],
        out_shape=[
            jax.ShapeDtypeStruct((N_PAD, 64), jnp.float32),
            jax.ShapeDtypeStruct((N_PAD, 64), jnp.float32),
        ],
    )(s1, part1, part1)


def _tc3(h1, part2, dinv, w2, b2, w3, b3):
    pa, pb = _part_specs(64)

    def body(h1_ref, pa_ref, pb_ref, dinv_ref, w2_ref, b2_ref, w3_ref, b3_ref,
             s3_ref, p3_ref):
        h1 = h1_ref[...]
        agg2 = (pa_ref[...] + pb_ref[...]) * dinv_ref[...]
        w2 = w2_ref[...]
        h2 = _l2relu(_mm(h1, w2[:64]) + _mm(agg2, w2[64:]) + b2_ref[0:1, :])
        w3 = w3_ref[...]
        s3_ref[...] = _mm(h2, w3[:256]) + b3_ref[0:1, :]
        p3_ref[...] = _mm(h2, w3[256:])

    return pl.pallas_call(
        body,
        grid=(NB,),
        in_specs=[_row_spec(64), pa, pb, _row_spec(64),
                  _full_spec((128, 256)), _full_spec((8, 256)),
                  _full_spec((512, 32)), _full_spec((8, 32))],
        out_specs=[_row_spec(32), _row_spec(32)],
        out_shape=[
            jax.ShapeDtypeStruct((N_PAD, 32), jnp.float32),
            jax.ShapeDtypeStruct((N_PAD, 32), jnp.float32),
        ],
    )(h1, part2, part2, dinv, w2, b2, w3, b3)


def _tc4(s3, part3, dinv):
    pa, pb = _part_specs(32)

    def body(s3_ref, pa_ref, pb_ref, dinv_ref, h3_ref):
        agg3 = (pa_ref[...] + pb_ref[...]) * dinv_ref[:, :32]
        h3_ref[...] = _l2relu(s3_ref[...] + agg3)

    return pl.pallas_call(
        body,
        grid=(NB,),
        in_specs=[_row_spec(32), pa, pb, _row_spec(64)],
        out_specs=_row_spec(32),
        out_shape=jax.ShapeDtypeStruct((N_PAD, 32), jnp.float32),
    )(s3, part3, part3, dinv)


def _tc5(h3, part4, dinv, w4, b4, w5, b5):
    pa, pb = _part_specs(32)

    def body(h3_ref, pa_ref, pb_ref, dinv_ref, w4_ref, b4_ref, w5_ref, b5_ref,
             s5_ref, p5_ref):
        h3 = h3_ref[...]
        agg4 = (pa_ref[...] + pb_ref[...]) * dinv_ref[:, :32]
        w4 = w4_ref[...]
        h4 = _l2relu(_mm(h3, w4[:32]) + _mm(agg4, w4[32:]) + b4_ref[0:1, :])
        w5 = w5_ref[...]
        s5_ref[...] = _mm(h4, w5[:64]) + b5_ref[0:1, :]
        p5_ref[...] = _mm(h4, w5[64:])

    return pl.pallas_call(
        body,
        grid=(NB,),
        in_specs=[_row_spec(32), pa, pb, _row_spec(64),
                  _full_spec((64, 64)), _full_spec((8, 64)),
                  _full_spec((128, 64)), _full_spec((8, 64))],
        out_specs=[_row_spec(64), _row_spec(64)],
        out_shape=[
            jax.ShapeDtypeStruct((N_PAD, 64), jnp.float32),
            jax.ShapeDtypeStruct((N_PAD, 64), jnp.float32),
        ],
    )(h3, part4, part4, dinv, w4, b4, w5, b5)


def _tc6(s5, part5, dinv, ib, wd, bd):
    pa, pb = _part_specs(64)

    def body(s5_ref, pa_ref, pb_ref, dinv_ref, ib_ref, wd_ref, bd_ref,
             out_ref, psum, cnt):
        k = pl.program_id(0)

        @pl.when(k == 0)
        def _():
            psum[...] = jnp.zeros((G, 64), jnp.float32)
            cnt[...] = jnp.zeros((G, 16), jnp.float32)

        agg5 = (pa_ref[...] + pb_ref[...]) * dinv_ref[...]
        h5 = _l2relu(s5_ref[...] + agg5)
        m = (ib_ref[...] == lax.broadcasted_iota(jnp.int32, (RBLK, G), 1))
        m = m.astype(jnp.float32)
        cT = (((0,), (0,)), ((), ()))
        psum[...] += lax.dot_general(
            m, h5, cT, precision=lax.Precision.HIGHEST,
            preferred_element_type=jnp.float32)
        cnt[...] += lax.dot_general(
            m, jnp.ones((RBLK, 16), jnp.float32), cT,
            precision=lax.Precision.HIGHEST,
            preferred_element_type=jnp.float32)

        @pl.when(k == NB - 1)
        def _():
            pooled = psum[...] * (1.0 / jnp.maximum(cnt[:, 0:1], 1.0))
            out_ref[...] = jnp.tanh(_mm(pooled, wd_ref[...]) + bd_ref[0:1, :])

    return pl.pallas_call(
        body,
        grid=(NB,),
        in_specs=[_row_spec(64), pa, pb, _row_spec(64), _row_spec(64),
                  _full_spec((64, 16)), _full_spec((8, 16))],
        out_specs=_full_spec((G, 16)),
        out_shape=jax.ShapeDtypeStruct((G, 16), jnp.float32),
        scratch_shapes=[
            pltpu.VMEM((G, 64), jnp.float32),
            pltpu.VMEM((G, 16), jnp.float32),
        ],
    )(s5, part5, part5, dinv, ib, wd, bd)


def _pad_bias(b, d):
    return jnp.broadcast_to(b[None, :], (8, d))


def kernel(x, edge_index, i, W1, b1, W2, b2, W3, b3, W4, b4, W5, b5, Wd, bd):
    f32 = jnp.float32
    xp = jnp.pad(x, ((0, N_PAD - N), (0, 0)))
    src = jnp.concatenate(
        [edge_index[0], jnp.zeros((E_PAD - E,), jnp.int32)]).reshape(-1, CH)
    dst = jnp.concatenate(
        [edge_index[1], jnp.full((E_PAD - E,), N, jnp.int32)]).reshape(-1, CH)
    ip = jnp.concatenate([i, jnp.full((N_PAD - N,), G, jnp.int32)])
    ib = jnp.broadcast_to(ip[:, None], (N_PAD, G))
    wd16 = jnp.pad(Wd, ((0, 0), (0, 6)))
    bd16 = _pad_bias(jnp.pad(bd, (0, 6)), 16)

    s1, pd1 = _tc1(xp, W1, _pad_bias(b1, 64))
    part1 = _sc_aggregate(pd1, src, dst)
    h1, dinv = _tc2(s1, part1)

    part2 = _sc_aggregate(h1, src, dst)
    s3, p3 = _tc3(h1, part2, dinv, W2, _pad_bias(b2, 256), W3, _pad_bias(b3, 32))

    part3 = _sc_aggregate(p3, src, dst)
    h3 = _tc4(s3, part3, dinv)

    part4 = _sc_aggregate(h3, src, dst)
    s5, p5 = _tc5(h3, part4, dinv, W4, _pad_bias(b4, 64), W5, _pad_bias(b5, 64))

    part5 = _sc_aggregate(p5, src, dst)
    out16 = _tc6(s5, part5, dinv, ib, wd16, bd16)
    return out16[:, :10].astype(f32)


# speedup vs baseline: 5.8442x; 1.0110x over previous
import functools

import jax
import jax.numpy as jnp
from jax import lax
from jax.experimental import pallas as pl
from jax.experimental.pallas import tpu as pltpu
from jax.experimental.pallas import tpu_sc as plsc

N = 10000
E = 320000
G = 64
N_PAD = 10240
E_PAD = 327680
NC = 2
NS = 16
CH = 128
EPT = E_PAD // (NC * NS)
NCHUNK = EPT // CH
RPT = N_PAD // NS
RBLK = 1024
NB = N_PAD // RBLK


_K_FOR_D = {80: 5, 64: 8, 32: 16}


@functools.lru_cache(maxsize=None)
def _make_sc_agg(d):
    K = _K_FOR_D[d]
    NGRP = NCHUNK // K
    mesh = plsc.VectorSubcoreMesh(
        core_axis_name="c", subcore_axis_name="s",
        num_cores=NC, num_subcores=NS)

    @functools.partial(
        pl.kernel,
        mesh=mesh,
        compiler_params=pltpu.CompilerParams(use_tc_tiling_on_sc=False),
        out_type=jax.ShapeDtypeStruct((NC * N_PAD, d), jnp.float32),
        scratch_types=(
            [pltpu.VMEM((NCHUNK, CH), jnp.int32)]
            + [pltpu.VMEM((NCHUNK, CH), jnp.int32)]
            + [pltpu.VMEM((CH, d), jnp.float32) for _ in range(K)]
            + [pltpu.VMEM_SHARED((N_PAD, d), jnp.float32)]
            + [pltpu.SemaphoreType.DMA, pltpu.SemaphoreType.DMA]
        ),
    )
    def agg(vals, src2, dst2, zeros, out, *scr):
        src_all = scr[0]
        dst_all = scr[1]
        rows = scr[2:2 + K]
        acc = scr[2 + K]
        gsem = scr[3 + K]
        ssem = scr[4 + K]
        c = lax.axis_index("c")
        s = lax.axis_index("s")
        pltpu.sync_copy(zeros, acc.at[pl.ds(s * RPT, RPT)])
        row0 = (c * NS + s) * NCHUNK
        pltpu.sync_copy(src2.at[pl.ds(row0, NCHUNK)], src_all)
        pltpu.sync_copy(dst2.at[pl.ds(row0, NCHUNK)], dst_all)
        plsc.subcore_barrier()

        def group(j, carry):
            g0 = j * K
            for b in range(K):
                pltpu.async_copy(vals.at[src_all.at[g0 + b]], rows[b], gsem)
            for b in range(K):
                pltpu.make_async_copy(
                    vals.at[src_all.at[g0 + b]], rows[b], gsem).wait()
            for b in range(K):
                pltpu.async_copy(
                    rows[b], acc.at[dst_all.at[g0 + b]], ssem, add=True)
            for b in range(K):
                pltpu.make_async_copy(
                    rows[b], acc.at[dst_all.at[g0 + b]], ssem).wait()
            return carry

        lax.fori_loop(0, NGRP, group, 0)
        plsc.subcore_barrier()
        pltpu.sync_copy(
            acc.at[pl.ds(s * RPT, RPT)],
            out.at[pl.ds(c * N_PAD + s * RPT, RPT)],
        )

    return agg


def _sc_aggregate(vals, src, dst):
    d = vals.shape[1]
    zeros = jnp.zeros((RPT, d), jnp.float32)
    return _make_sc_agg(d)(vals, src, dst, zeros)


def _mm(a, b):
    return lax.dot_general(
        a, b, (((1,), (0,)), ((), ())),
        precision=lax.Precision.HIGHEST,
        preferred_element_type=jnp.float32,
    )


def _l2relu(t):
    nrm = lax.rsqrt(jnp.maximum(jnp.sum(t * t, -1, keepdims=True), 1e-12))
    return jnp.maximum(t * nrm, 0.0)


def _row_spec(d):
    return pl.BlockSpec((RBLK, d), lambda k: (k, 0))


def _part_specs(d):
    return (
        pl.BlockSpec((RBLK, d), lambda k: (k, 0)),
        pl.BlockSpec((RBLK, d), lambda k: (k + NB, 0)),
    )


def _full_spec(shape):
    nd = len(shape)
    return pl.BlockSpec(shape, lambda k: (0,) * nd)


def _tc1_body(x_ref, w_ref, b_ref, s1_ref, pd1_ref):
    x = x_ref[...]
    w = w_ref[...]
    s1_ref[...] = _mm(x, w[:128]) + b_ref[0:1, :]
    p = _mm(x, w[128:])
    onescol = (lax.broadcasted_iota(jnp.int32, (RBLK, 16), 1) == 0).astype(jnp.float32)
    pd1_ref[...] = jnp.concatenate([p, onescol], axis=1)


def _tc1(x, w1, b1):
    return pl.pallas_call(
        _tc1_body,
        grid=(NB,),
        in_specs=[_row_spec(128), _full_spec((256, 64)), _full_spec((8, 64))],
        out_specs=[_row_spec(64), _row_spec(80)],
        out_shape=[
            jax.ShapeDtypeStruct((N_PAD, 64), jnp.float32),
            jax.ShapeDtypeStruct((N_PAD, 80), jnp.float32),
        ],
    )(x, w1, b1)


def _tc2_body(s1_ref, pa_ref, pb_ref, h1_ref, dinv_ref):
    p = pa_ref[...] + pb_ref[...]
    deg = jnp.sum(p[:, 64:80], -1, keepdims=True)
    dinv = 1.0 / jnp.maximum(deg, 1.0)
    t = s1_ref[...] + p[:, :64] * dinv
    h1_ref[...] = _l2relu(t)
    dinv_ref[...] = jnp.broadcast_to(dinv, (RBLK, 64))


def _tc2(s1, part1):
    pa, pb = _part_specs(80)
    return pl.pallas_call(
        _tc2_body,
        grid=(NB,),
        in_specs=[_row_spec(64), pa, pb],
        out_specs=[_row_spec(64), _row_spec(64)],
        out_shape=[
            jax.ShapeDtypeStruct((N_PAD, 64), jnp.float32),
            jax.ShapeDtypeStruct((N_PAD, 64), jnp.float32),
        ],
    )(s1, part1, part1)


def _tc3(h1, part2, dinv, w2, b2, w3, b3):
    pa, pb = _part_specs(64)

    def body(h1_ref, pa_ref, pb_ref, dinv_ref, w2_ref, b2_ref, w3_ref, b3_ref,
             s3_ref, p3_ref):
        h1 = h1_ref[...]
        agg2 = (pa_ref[...] + pb_ref[...]) * dinv_ref[...]
        w2 = w2_ref[...]
        h2 = _l2relu(_mm(h1, w2[:64]) + _mm(agg2, w2[64:]) + b2_ref[0:1, :])
        w3 = w3_ref[...]
        s3_ref[...] = _mm(h2, w3[:256]) + b3_ref[0:1, :]
        p3_ref[...] = _mm(h2, w3[256:])

    return pl.pallas_call(
        body,
        grid=(NB,),
        in_specs=[_row_spec(64), pa, pb, _row_spec(64),
                  _full_spec((128, 256)), _full_spec((8, 256)),
                  _full_spec((512, 32)), _full_spec((8, 32))],
        out_specs=[_row_spec(32), _row_spec(32)],
        out_shape=[
            jax.ShapeDtypeStruct((N_PAD, 32), jnp.float32),
            jax.ShapeDtypeStruct((N_PAD, 32), jnp.float32),
        ],
    )(h1, part2, part2, dinv, w2, b2, w3, b3)


def _tc4(s3, part3, dinv):
    pa, pb = _part_specs(32)

    def body(s3_ref, pa_ref, pb_ref, dinv_ref, h3_ref):
        agg3 = (pa_ref[...] + pb_ref[...]) * dinv_ref[:, :32]
        h3_ref[...] = _l2relu(s3_ref[...] + agg3)

    return pl.pallas_call(
        body,
        grid=(NB,),
        in_specs=[_row_spec(32), pa, pb, _row_spec(64)],
        out_specs=_row_spec(32),
        out_shape=jax.ShapeDtypeStruct((N_PAD, 32), jnp.float32),
    )(s3, part3, part3, dinv)


def _tc5(h3, part4, dinv, w4, b4, w5, b5):
    pa, pb = _part_specs(32)

    def body(h3_ref, pa_ref, pb_ref, dinv_ref, w4_ref, b4_ref, w5_ref, b5_ref,
             s5_ref, p5_ref):
        h3 = h3_ref[...]
        agg4 = (pa_ref[...] + pb_ref[...]) * dinv_ref[:, :32]
        w4 = w4_ref[...]
        h4 = _l2relu(_mm(h3, w4[:32]) + _mm(agg4, w4[32:]) + b4_ref[0:1, :])
        w5 = w5_ref[...]
        s5_ref[...] = _mm(h4, w5[:64]) + b5_ref[0:1, :]
        p5_ref[...] = _mm(h4, w5[64:])

    return pl.pallas_call(
        body,
        grid=(NB,),
        in_specs=[_row_spec(32), pa, pb, _row_spec(64),
                  _full_spec((64, 64)), _full_spec((8, 64)),
                  _full_spec((128, 64)), _full_spec((8, 64))],
        out_specs=[_row_spec(64), _row_spec(64)],
        out_shape=[
            jax.ShapeDtypeStruct((N_PAD, 64), jnp.float32),
            jax.ShapeDtypeStruct((N_PAD, 64), jnp.float32),
        ],
    )(h3, part4, part4, dinv, w4, b4, w5, b5)


def _tc6(s5, part5, dinv, ib, wd, bd):
    pa, pb = _part_specs(64)

    def body(s5_ref, pa_ref, pb_ref, dinv_ref, ib_ref, wd_ref, bd_ref,
             out_ref, psum, cnt):
        k = pl.program_id(0)

        @pl.when(k == 0)
        def _():
            psum[...] = jnp.zeros((G, 64), jnp.float32)
            cnt[...] = jnp.zeros((G, 16), jnp.float32)

        agg5 = (pa_ref[...] + pb_ref[...]) * dinv_ref[...]
        h5 = _l2relu(s5_ref[...] + agg5)
        m = (ib_ref[...] == lax.broadcasted_iota(jnp.int32, (RBLK, G), 1))
        m = m.astype(jnp.float32)
        cT = (((0,), (0,)), ((), ()))
        psum[...] += lax.dot_general(
            m, h5, cT, precision=lax.Precision.HIGHEST,
            preferred_element_type=jnp.float32)
        cnt[...] += lax.dot_general(
            m, jnp.ones((RBLK, 16), jnp.float32), cT,
            precision=lax.Precision.HIGHEST,
            preferred_element_type=jnp.float32)

        @pl.when(k == NB - 1)
        def _():
            pooled = psum[...] * (1.0 / jnp.maximum(cnt[:, 0:1], 1.0))
            out_ref[...] = jnp.tanh(_mm(pooled, wd_ref[...]) + bd_ref[0:1, :])

    return pl.pallas_call(
        body,
        grid=(NB,),
        in_specs=[_row_spec(64), pa, pb, _row_spec(64), _row_spec(64),
                  _full_spec((64, 16)), _full_spec((8, 16))],
        out_specs=_full_spec((G, 16)),
        out_shape=jax.ShapeDtypeStruct((G, 16), jnp.float32),
        scratch_shapes=[
            pltpu.VMEM((G, 64), jnp.float32),
            pltpu.VMEM((G, 16), jnp.float32),
        ],
    )(s5, part5, part5, dinv, ib, wd, bd)


def _pad_bias(b, d):
    return jnp.broadcast_to(b[None, :], (8, d))


def kernel(x, edge_index, i, W1, b1, W2, b2, W3, b3, W4, b4, W5, b5, Wd, bd):
    f32 = jnp.float32
    xp = jnp.pad(x, ((0, N_PAD - N), (0, 0)))
    src = jnp.concatenate(
        [edge_index[0], jnp.zeros((E_PAD - E,), jnp.int32)]).reshape(-1, CH)
    dst = jnp.concatenate(
        [edge_index[1], jnp.full((E_PAD - E,), N, jnp.int32)]).reshape(-1, CH)
    ip = jnp.concatenate([i, jnp.full((N_PAD - N,), G, jnp.int32)])
    ib = jnp.broadcast_to(ip[:, None], (N_PAD, G))
    wd16 = jnp.pad(Wd, ((0, 0), (0, 6)))
    bd16 = _pad_bias(jnp.pad(bd, (0, 6)), 16)

    s1, pd1 = _tc1(xp, W1, _pad_bias(b1, 64))
    part1 = _sc_aggregate(pd1, src, dst)
    h1, dinv = _tc2(s1, part1)

    part2 = _sc_aggregate(h1, src, dst)
    s3, p3 = _tc3(h1, part2, dinv, W2, _pad_bias(b2, 256), W3, _pad_bias(b3, 32))

    part3 = _sc_aggregate(p3, src, dst)
    h3 = _tc4(s3, part3, dinv)

    part4 = _sc_aggregate(h3, src, dst)
    s5, p5 = _tc5(h3, part4, dinv, W4, _pad_bias(b4, 64), W5, _pad_bias(b5, 64))

    part5 = _sc_aggregate(p5, src, dst)
    out16 = _tc6(s5, part5, dinv, ib, wd16, bd16)
    return out16[:, :10].astype(f32)


# speedup vs baseline: 6.0676x; 1.0382x over previous
import functools

import jax
import jax.numpy as jnp
from jax import lax
from jax.experimental import pallas as pl
from jax.experimental.pallas import tpu as pltpu
from jax.experimental.pallas import tpu_sc as plsc

N = 10000
E = 320000
G = 64
N_PAD = 10240
E_PAD = 327680
NC = 2
NS = 16
CH = 128
EPT = E_PAD // (NC * NS)
NCHUNK = EPT // CH
RPT = N_PAD // NS
RBLK = 1024
NB = N_PAD // RBLK


_K_FOR_D = {80: 2, 64: 4, 32: 8}


@functools.lru_cache(maxsize=None)
def _make_sc_agg(d):
    K = _K_FOR_D[d]
    NGRP = NCHUNK // K
    NPAIR = (NGRP - 2) // 2
    mesh = plsc.VectorSubcoreMesh(
        core_axis_name="c", subcore_axis_name="s",
        num_cores=NC, num_subcores=NS)

    @functools.partial(
        pl.kernel,
        mesh=mesh,
        compiler_params=pltpu.CompilerParams(use_tc_tiling_on_sc=False),
        out_type=jax.ShapeDtypeStruct((NC * N_PAD, d), jnp.float32),
        scratch_types=(
            [pltpu.VMEM((NCHUNK, CH), jnp.int32)]
            + [pltpu.VMEM((NCHUNK, CH), jnp.int32)]
            + [pltpu.VMEM((CH, d), jnp.float32) for _ in range(2 * K)]
            + [pltpu.VMEM_SHARED((N_PAD, d), jnp.float32)]
            + [pltpu.SemaphoreType.DMA for _ in range(4)]
        ),
    )
    def agg(vals, src2, dst2, zeros, out, *scr):
        src_all = scr[0]
        dst_all = scr[1]
        rows_a = scr[2:2 + K]
        rows_b = scr[2 + K:2 + 2 * K]
        acc = scr[2 + 2 * K]
        gs_a, gs_b, ss_a, ss_b = scr[3 + 2 * K:7 + 2 * K]
        c = lax.axis_index("c")
        s = lax.axis_index("s")
        pltpu.sync_copy(zeros, acc.at[pl.ds(s * RPT, RPT)])
        row0 = (c * NS + s) * NCHUNK
        pltpu.sync_copy(src2.at[pl.ds(row0, NCHUNK)], src_all)
        pltpu.sync_copy(dst2.at[pl.ds(row0, NCHUNK)], dst_all)
        plsc.subcore_barrier()

        def fire_g(rows, gsem, g):
            for b in range(K):
                pltpu.async_copy(vals.at[src_all.at[g * K + b]], rows[b], gsem)

        def wait_g(rows, gsem, g):
            for b in range(K):
                pltpu.make_async_copy(
                    vals.at[src_all.at[g * K + b]], rows[b], gsem).wait()

        def fire_s(rows, ssem, g):
            for b in range(K):
                pltpu.async_copy(
                    rows[b], acc.at[dst_all.at[g * K + b]], ssem, add=True)

        def drain_s(rows, ssem, g):
            for b in range(K):
                pltpu.make_async_copy(
                    rows[b], acc.at[dst_all.at[g * K + b]], ssem).wait()

        fire_g(rows_a, gs_a, 0)
        wait_g(rows_a, gs_a, 0)
        fire_g(rows_b, gs_b, 1)
        fire_s(rows_a, ss_a, 0)

        def pair(j, carry):
            gb = 2 * j + 1
            ga = 2 * j + 2
            wait_g(rows_b, gs_b, gb)
            drain_s(rows_a, ss_a, ga - 2)
            fire_g(rows_a, gs_a, ga)
            fire_s(rows_b, ss_b, gb)
            wait_g(rows_a, gs_a, ga)
            drain_s(rows_b, ss_b, gb)
            fire_g(rows_b, gs_b, ga + 1)
            fire_s(rows_a, ss_a, ga)
            return carry

        lax.fori_loop(0, NPAIR, pair, 0)
        wait_g(rows_b, gs_b, NGRP - 1)
        drain_s(rows_a, ss_a, NGRP - 2)
        fire_s(rows_b, ss_b, NGRP - 1)
        drain_s(rows_b, ss_b, NGRP - 1)

        plsc.subcore_barrier()
        pltpu.sync_copy(
            acc.at[pl.ds(s * RPT, RPT)],
            out.at[pl.ds(c * N_PAD + s * RPT, RPT)],
        )

    return agg


def _sc_aggregate(vals, src, dst):
    d = vals.shape[1]
    zeros = jnp.zeros((RPT, d), jnp.float32)
    return _make_sc_agg(d)(vals, src, dst, zeros)


def _mm(a, b):
    return lax.dot_general(
        a, b, (((1,), (0,)), ((), ())),
        precision=lax.Precision.HIGHEST,
        preferred_element_type=jnp.float32,
    )


def _l2relu(t):
    nrm = lax.rsqrt(jnp.maximum(jnp.sum(t * t, -1, keepdims=True), 1e-12))
    return jnp.maximum(t * nrm, 0.0)


def _row_spec(d):
    return pl.BlockSpec((RBLK, d), lambda k: (k, 0))


def _part_specs(d):
    return (
        pl.BlockSpec((RBLK, d), lambda k: (k, 0)),
        pl.BlockSpec((RBLK, d), lambda k: (k + NB, 0)),
    )


def _full_spec(shape):
    nd = len(shape)
    return pl.BlockSpec(shape, lambda k: (0,) * nd)


def _tc1_body(x_ref, w_ref, b_ref, s1_ref, pd1_ref):
    x = x_ref[...]
    w = w_ref[...]
    s1_ref[...] = _mm(x, w[:128]) + b_ref[0:1, :]
    p = _mm(x, w[128:])
    onescol = (lax.broadcasted_iota(jnp.int32, (RBLK, 16), 1) == 0).astype(jnp.float32)
    pd1_ref[...] = jnp.concatenate([p, onescol], axis=1)


def _tc1(x, w1, b1):
    return pl.pallas_call(
        _tc1_body,
        grid=(NB,),
        in_specs=[_row_spec(128), _full_spec((256, 64)), _full_spec((8, 64))],
        out_specs=[_row_spec(64), _row_spec(80)],
        out_shape=[
            jax.ShapeDtypeStruct((N_PAD, 64), jnp.float32),
            jax.ShapeDtypeStruct((N_PAD, 80), jnp.float32),
        ],
    )(x, w1, b1)


def _tc2_body(s1_ref, pa_ref, pb_ref, h1_ref, dinv_ref):
    p = pa_ref[...] + pb_ref[...]
    deg = jnp.sum(p[:, 64:80], -1, keepdims=True)
    dinv = 1.0 / jnp.maximum(deg, 1.0)
    t = s1_ref[...] + p[:, :64] * dinv
    h1_ref[...] = _l2relu(t)
    dinv_ref[...] = jnp.broadcast_to(dinv, (RBLK, 64))


def _tc2(s1, part1):
    pa, pb = _part_specs(80)
    return pl.pallas_call(
        _tc2_body,
        grid=(NB,),
        in_specs=[_row_spec(64), pa, pb],
        out_specs=[_row_spec(64), _row_spec(64)],
        out_shape=[
            jax.ShapeDtypeStruct((N_PAD, 64), jnp.float32),
            jax.ShapeDtypeStruct((N_PAD, 64), jnp.float32),
        ],
    )(s1, part1, part1)


def _tc3(h1, part2, dinv, w2, b2, w3, b3):
    pa, pb = _part_specs(64)

    def body(h1_ref, pa_ref, pb_ref, dinv_ref, w2_ref, b2_ref, w3_ref, b3_ref,
             s3_ref, p3_ref):
        h1 = h1_ref[...]
        agg2 = (pa_ref[...] + pb_ref[...]) * dinv_ref[...]
        w2 = w2_ref[...]
        h2 = _l2relu(_mm(h1, w2[:64]) + _mm(agg2, w2[64:]) + b2_ref[0:1, :])
        w3 = w3_ref[...]
        s3_ref[...] = _mm(h2, w3[:256]) + b3_ref[0:1, :]
        p3_ref[...] = _mm(h2, w3[256:])

    return pl.pallas_call(
        body,
        grid=(NB,),
        in_specs=[_row_spec(64), pa, pb, _row_spec(64),
                  _full_spec((128, 256)), _full_spec((8, 256)),
                  _full_spec((512, 32)), _full_spec((8, 32))],
        out_specs=[_row_spec(32), _row_spec(32)],
        out_shape=[
            jax.ShapeDtypeStruct((N_PAD, 32), jnp.float32),
            jax.ShapeDtypeStruct((N_PAD, 32), jnp.float32),
        ],
    )(h1, part2, part2, dinv, w2, b2, w3, b3)


def _tc4(s3, part3, dinv):
    pa, pb = _part_specs(32)

    def body(s3_ref, pa_ref, pb_ref, dinv_ref, h3_ref):
        agg3 = (pa_ref[...] + pb_ref[...]) * dinv_ref[:, :32]
        h3_ref[...] = _l2relu(s3_ref[...] + agg3)

    return pl.pallas_call(
        body,
        grid=(NB,),
        in_specs=[_row_spec(32), pa, pb, _row_spec(64)],
        out_specs=_row_spec(32),
        out_shape=jax.ShapeDtypeStruct((N_PAD, 32), jnp.float32),
    )(s3, part3, part3, dinv)


def _tc5(h3, part4, dinv, w4, b4, w5, b5):
    pa, pb = _part_specs(32)

    def body(h3_ref, pa_ref, pb_ref, dinv_ref, w4_ref, b4_ref, w5_ref, b5_ref,
             s5_ref, p5_ref):
        h3 = h3_ref[...]
        agg4 = (pa_ref[...] + pb_ref[...]) * dinv_ref[:, :32]
        w4 = w4_ref[...]
        h4 = _l2relu(_mm(h3, w4[:32]) + _mm(agg4, w4[32:]) + b4_ref[0:1, :])
        w5 = w5_ref[...]
        s5_ref[...] = _mm(h4, w5[:64]) + b5_ref[0:1, :]
        p5_ref[...] = _mm(h4, w5[64:])

    return pl.pallas_call(
        body,
        grid=(NB,),
        in_specs=[_row_spec(32), pa, pb, _row_spec(64),
                  _full_spec((64, 64)), _full_spec((8, 64)),
                  _full_spec((128, 64)), _full_spec((8, 64))],
        out_specs=[_row_spec(64), _row_spec(64)],
        out_shape=[
            jax.ShapeDtypeStruct((N_PAD, 64), jnp.float32),
            jax.ShapeDtypeStruct((N_PAD, 64), jnp.float32),
        ],
    )(h3, part4, part4, dinv, w4, b4, w5, b5)


def _tc6(s5, part5, dinv, ib, wd, bd):
    pa, pb = _part_specs(64)

    def body(s5_ref, pa_ref, pb_ref, dinv_ref, ib_ref, wd_ref, bd_ref,
             out_ref, psum, cnt):
        k = pl.program_id(0)

        @pl.when(k == 0)
        def _():
            psum[...] = jnp.zeros((G, 64), jnp.float32)
            cnt[...] = jnp.zeros((G, 16), jnp.float32)

        agg5 = (pa_ref[...] + pb_ref[...]) * dinv_ref[...]
        h5 = _l2relu(s5_ref[...] + agg5)
        m = (ib_ref[...] == lax.broadcasted_iota(jnp.int32, (RBLK, G), 1))
        m = m.astype(jnp.float32)
        cT = (((0,), (0,)), ((), ()))
        psum[...] += lax.dot_general(
            m, h5, cT, precision=lax.Precision.HIGHEST,
            preferred_element_type=jnp.float32)
        cnt[...] += lax.dot_general(
            m, jnp.ones((RBLK, 16), jnp.float32), cT,
            precision=lax.Precision.HIGHEST,
            preferred_element_type=jnp.float32)

        @pl.when(k == NB - 1)
        def _():
            pooled = psum[...] * (1.0 / jnp.maximum(cnt[:, 0:1], 1.0))
            out_ref[...] = jnp.tanh(_mm(pooled, wd_ref[...]) + bd_ref[0:1, :])

    return pl.pallas_call(
        body,
        grid=(NB,),
        in_specs=[_row_spec(64), pa, pb, _row_spec(64), _row_spec(64),
                  _full_spec((64, 16)), _full_spec((8, 16))],
        out_specs=_full_spec((G, 16)),
        out_shape=jax.ShapeDtypeStruct((G, 16), jnp.float32),
        scratch_shapes=[
            pltpu.VMEM((G, 64), jnp.float32),
            pltpu.VMEM((G, 16), jnp.float32),
        ],
    )(s5, part5, part5, dinv, ib, wd, bd)


def _pad_bias(b, d):
    return jnp.broadcast_to(b[None, :], (8, d))


def kernel(x, edge_index, i, W1, b1, W2, b2, W3, b3, W4, b4, W5, b5, Wd, bd):
    f32 = jnp.float32
    xp = jnp.pad(x, ((0, N_PAD - N), (0, 0)))
    src = jnp.concatenate(
        [edge_index[0], jnp.zeros((E_PAD - E,), jnp.int32)]).reshape(-1, CH)
    dst = jnp.concatenate(
        [edge_index[1], jnp.full((E_PAD - E,), N, jnp.int32)]).reshape(-1, CH)
    ip = jnp.concatenate([i, jnp.full((N_PAD - N,), G, jnp.int32)])
    ib = jnp.broadcast_to(ip[:, None], (N_PAD, G))
    wd16 = jnp.pad(Wd, ((0, 0), (0, 6)))
    bd16 = _pad_bias(jnp.pad(bd, (0, 6)), 16)

    s1, pd1 = _tc1(xp, W1, _pad_bias(b1, 64))
    part1 = _sc_aggregate(pd1, src, dst)
    h1, dinv = _tc2(s1, part1)

    part2 = _sc_aggregate(h1, src, dst)
    s3, p3 = _tc3(h1, part2, dinv, W2, _pad_bias(b2, 256), W3, _pad_bias(b3, 32))

    part3 = _sc_aggregate(p3, src, dst)
    h3 = _tc4(s3, part3, dinv)

    part4 = _sc_aggregate(h3, src, dst)
    s5, p5 = _tc5(h3, part4, dinv, W4, _pad_bias(b4, 64), W5, _pad_bias(b5, 64))

    part5 = _sc_aggregate(p5, src, dst)
    out16 = _tc6(s5, part5, dinv, ib, wd16, bd16)
    return out16[:, :10].astype(f32)


# speedup vs baseline: 6.9122x; 1.1392x over previous
import functools

import jax
import jax.numpy as jnp
from jax import lax
from jax.experimental import pallas as pl
from jax.experimental.pallas import tpu as pltpu
from jax.experimental.pallas import tpu_sc as plsc

N = 10000
E = 320000
G = 64
N_PAD = 10240
E_PAD = 327680
NC = 2
NS = 16
CH = 128
EPT = E_PAD // (NC * NS)
NCHUNK = EPT // CH
RPT = N_PAD // NS
RBLK = 1024
NB = N_PAD // RBLK


_K_FOR_D = {64: 4, 32: 8}


@functools.lru_cache(maxsize=None)
def _make_sc_agg(d):
    K = _K_FOR_D[d]
    NGRP = NCHUNK // K
    NPAIR = (NGRP - 2) // 2
    mesh = plsc.VectorSubcoreMesh(
        core_axis_name="c", subcore_axis_name="s",
        num_cores=NC, num_subcores=NS)

    @functools.partial(
        pl.kernel,
        mesh=mesh,
        compiler_params=pltpu.CompilerParams(use_tc_tiling_on_sc=False),
        out_type=jax.ShapeDtypeStruct((NC * N_PAD, d), jnp.float32),
        scratch_types=(
            [pltpu.VMEM((NCHUNK, CH), jnp.int32)]
            + [pltpu.VMEM((NCHUNK, CH), jnp.int32)]
            + [pltpu.VMEM((CH, d), jnp.float32) for _ in range(2 * K)]
            + [pltpu.VMEM_SHARED((N_PAD, d), jnp.float32)]
            + [pltpu.SemaphoreType.DMA for _ in range(4)]
        ),
    )
    def agg(vals, src2, dst2, zeros, out, *scr):
        src_all = scr[0]
        dst_all = scr[1]
        rows_a = scr[2:2 + K]
        rows_b = scr[2 + K:2 + 2 * K]
        acc = scr[2 + 2 * K]
        gs_a, gs_b, ss_a, ss_b = scr[3 + 2 * K:7 + 2 * K]
        c = lax.axis_index("c")
        s = lax.axis_index("s")
        pltpu.sync_copy(zeros, acc.at[pl.ds(s * RPT, RPT)])
        row0 = (c * NS + s) * NCHUNK
        pltpu.sync_copy(src2.at[pl.ds(row0, NCHUNK)], src_all)
        pltpu.sync_copy(dst2.at[pl.ds(row0, NCHUNK)], dst_all)
        plsc.subcore_barrier()

        def fire_g(rows, gsem, g):
            for b in range(K):
                pltpu.async_copy(vals.at[src_all.at[g * K + b]], rows[b], gsem)

        def wait_g(rows, gsem, g):
            for b in range(K):
                pltpu.make_async_copy(
                    vals.at[src_all.at[g * K + b]], rows[b], gsem).wait()

        def fire_s(rows, ssem, g):
            for b in range(K):
                pltpu.async_copy(
                    rows[b], acc.at[dst_all.at[g * K + b]], ssem, add=True)

        def drain_s(rows, ssem, g):
            for b in range(K):
                pltpu.make_async_copy(
                    rows[b], acc.at[dst_all.at[g * K + b]], ssem).wait()

        fire_g(rows_a, gs_a, 0)
        wait_g(rows_a, gs_a, 0)
        fire_g(rows_b, gs_b, 1)
        fire_s(rows_a, ss_a, 0)

        def pair(j, carry):
            gb = 2 * j + 1
            ga = 2 * j + 2
            wait_g(rows_b, gs_b, gb)
            drain_s(rows_a, ss_a, ga - 2)
            fire_g(rows_a, gs_a, ga)
            fire_s(rows_b, ss_b, gb)
            wait_g(rows_a, gs_a, ga)
            drain_s(rows_b, ss_b, gb)
            fire_g(rows_b, gs_b, ga + 1)
            fire_s(rows_a, ss_a, ga)
            return carry

        lax.fori_loop(0, NPAIR, pair, 0)
        wait_g(rows_b, gs_b, NGRP - 1)
        drain_s(rows_a, ss_a, NGRP - 2)
        fire_s(rows_b, ss_b, NGRP - 1)
        drain_s(rows_b, ss_b, NGRP - 1)

        plsc.subcore_barrier()
        pltpu.sync_copy(
            acc.at[pl.ds(s * RPT, RPT)],
            out.at[pl.ds(c * N_PAD + s * RPT, RPT)],
        )

    return agg


def _sc_aggregate(vals, src, dst):
    d = vals.shape[1]
    zeros = jnp.zeros((RPT, d), jnp.float32)
    return _make_sc_agg(d)(vals, src, dst, zeros)


DEG_W = 16
DEG_K = 8


def _make_sc_deg():
    mesh = plsc.VectorSubcoreMesh(
        core_axis_name="c", subcore_axis_name="s",
        num_cores=NC, num_subcores=NS)

    @functools.partial(
        pl.kernel,
        mesh=mesh,
        compiler_params=pltpu.CompilerParams(use_tc_tiling_on_sc=False),
        out_type=jax.ShapeDtypeStruct((NC * N_PAD, DEG_W), jnp.float32),
        scratch_types=(
            [pltpu.VMEM((NCHUNK, CH), jnp.int32)]
            + [pltpu.VMEM((CH, DEG_W), jnp.float32)]
            + [pltpu.VMEM_SHARED((N_PAD, DEG_W), jnp.float32)]
            + [pltpu.SemaphoreType.DMA]
        ),
    )
    def deg(dst2, ones, zeros, out, dst_all, ones_v, acc, ssem):
        c = lax.axis_index("c")
        s = lax.axis_index("s")
        pltpu.sync_copy(zeros, acc.at[pl.ds(s * RPT, RPT)])
        row0 = (c * NS + s) * NCHUNK
        pltpu.sync_copy(dst2.at[pl.ds(row0, NCHUNK)], dst_all)
        pltpu.sync_copy(ones, ones_v)
        plsc.subcore_barrier()

        def group(j, carry):
            g0 = j * DEG_K
            for b in range(DEG_K):
                pltpu.async_copy(
                    ones_v, acc.at[dst_all.at[g0 + b]], ssem, add=True)
            for b in range(DEG_K):
                pltpu.make_async_copy(
                    ones_v, acc.at[dst_all.at[g0 + b]], ssem).wait()
            return carry

        lax.fori_loop(0, NCHUNK // DEG_K, group, 0)
        plsc.subcore_barrier()
        pltpu.sync_copy(
            acc.at[pl.ds(s * RPT, RPT)],
            out.at[pl.ds(c * N_PAD + s * RPT, RPT)],
        )

    return deg


def _mm(a, b):
    return lax.dot_general(
        a, b, (((1,), (0,)), ((), ())),
        precision=lax.Precision.HIGHEST,
        preferred_element_type=jnp.float32,
    )


def _l2relu(t):
    nrm = lax.rsqrt(jnp.maximum(jnp.sum(t * t, -1, keepdims=True), 1e-12))
    return jnp.maximum(t * nrm, 0.0)


def _row_spec(d):
    return pl.BlockSpec((RBLK, d), lambda k: (k, 0))


def _part_specs(d):
    return (
        pl.BlockSpec((RBLK, d), lambda k: (k, 0)),
        pl.BlockSpec((RBLK, d), lambda k: (k + NB, 0)),
    )


def _full_spec(shape):
    nd = len(shape)
    return pl.BlockSpec(shape, lambda k: (0,) * nd)


def _tc1_body(x_ref, w_ref, b_ref, s1_ref, p1_ref):
    x = x_ref[...]
    w = w_ref[...]
    s1_ref[...] = _mm(x, w[:128]) + b_ref[0:1, :]
    p1_ref[...] = _mm(x, w[128:])


def _tc1(x, w1, b1):
    return pl.pallas_call(
        _tc1_body,
        grid=(NB,),
        in_specs=[_row_spec(128), _full_spec((256, 64)), _full_spec((8, 64))],
        out_specs=[_row_spec(64), _row_spec(64)],
        out_shape=[
            jax.ShapeDtypeStruct((N_PAD, 64), jnp.float32),
            jax.ShapeDtypeStruct((N_PAD, 64), jnp.float32),
        ],
    )(x, w1, b1)


def _tc2_body(s1_ref, pa_ref, pb_ref, da_ref, db_ref, h1_ref, dinv_ref):
    p = pa_ref[...] + pb_ref[...]
    deg = jnp.sum(da_ref[...] + db_ref[...], -1, keepdims=True)
    dinv = 1.0 / jnp.maximum(deg, 1.0)
    t = s1_ref[...] + p * dinv
    h1_ref[...] = _l2relu(t)
    dinv_ref[...] = jnp.broadcast_to(dinv, (RBLK, 64))


def _tc2(s1, part1, deg_part):
    pa, pb = _part_specs(64)
    da, db = _part_specs(DEG_W)
    return pl.pallas_call(
        _tc2_body,
        grid=(NB,),
        in_specs=[_row_spec(64), pa, pb, da, db],
        out_specs=[_row_spec(64), _row_spec(64)],
        out_shape=[
            jax.ShapeDtypeStruct((N_PAD, 64), jnp.float32),
            jax.ShapeDtypeStruct((N_PAD, 64), jnp.float32),
        ],
    )(s1, part1, part1, deg_part, deg_part)


def _tc3(h1, part2, dinv, w2, b2, w3, b3):
    pa, pb = _part_specs(64)

    def body(h1_ref, pa_ref, pb_ref, dinv_ref, w2_ref, b2_ref, w3_ref, b3_ref,
             s3_ref, p3_ref):
        h1 = h1_ref[...]
        agg2 = (pa_ref[...] + pb_ref[...]) * dinv_ref[...]
        w2 = w2_ref[...]
        h2 = _l2relu(_mm(h1, w2[:64]) + _mm(agg2, w2[64:]) + b2_ref[0:1, :])
        w3 = w3_ref[...]
        s3_ref[...] = _mm(h2, w3[:256]) + b3_ref[0:1, :]
        p3_ref[...] = _mm(h2, w3[256:])

    return pl.pallas_call(
        body,
        grid=(NB,),
        in_specs=[_row_spec(64), pa, pb, _row_spec(64),
                  _full_spec((128, 256)), _full_spec((8, 256)),
                  _full_spec((512, 32)), _full_spec((8, 32))],
        out_specs=[_row_spec(32), _row_spec(32)],
        out_shape=[
            jax.ShapeDtypeStruct((N_PAD, 32), jnp.float32),
            jax.ShapeDtypeStruct((N_PAD, 32), jnp.float32),
        ],
    )(h1, part2, part2, dinv, w2, b2, w3, b3)


def _tc4(s3, part3, dinv):
    pa, pb = _part_specs(32)

    def body(s3_ref, pa_ref, pb_ref, dinv_ref, h3_ref):
        agg3 = (pa_ref[...] + pb_ref[...]) * dinv_ref[:, :32]
        h3_ref[...] = _l2relu(s3_ref[...] + agg3)

    return pl.pallas_call(
        body,
        grid=(NB,),
        in_specs=[_row_spec(32), pa, pb, _row_spec(64)],
        out_specs=_row_spec(32),
        out_shape=jax.ShapeDtypeStruct((N_PAD, 32), jnp.float32),
    )(s3, part3, part3, dinv)


def _tc5(h3, part4, dinv, w4, b4, w5, b5):
    pa, pb = _part_specs(32)

    def body(h3_ref, pa_ref, pb_ref, dinv_ref, w4_ref, b4_ref, w5_ref, b5_ref,
             s5_ref, p5_ref):
        h3 = h3_ref[...]
        agg4 = (pa_ref[...] + pb_ref[...]) * dinv_ref[:, :32]
        w4 = w4_ref[...]
        h4 = _l2relu(_mm(h3, w4[:32]) + _mm(agg4, w4[32:]) + b4_ref[0:1, :])
        w5 = w5_ref[...]
        s5_ref[...] = _mm(h4, w5[:64]) + b5_ref[0:1, :]
        p5_ref[...] = _mm(h4, w5[64:])

    return pl.pallas_call(
        body,
        grid=(NB,),
        in_specs=[_row_spec(32), pa, pb, _row_spec(64),
                  _full_spec((64, 64)), _full_spec((8, 64)),
                  _full_spec((128, 64)), _full_spec((8, 64))],
        out_specs=[_row_spec(64), _row_spec(64)],
        out_shape=[
            jax.ShapeDtypeStruct((N_PAD, 64), jnp.float32),
            jax.ShapeDtypeStruct((N_PAD, 64), jnp.float32),
        ],
    )(h3, part4, part4, dinv, w4, b4, w5, b5)


def _tc6(s5, part5, dinv, ib, wd, bd):
    pa, pb = _part_specs(64)

    def body(s5_ref, pa_ref, pb_ref, dinv_ref, ib_ref, wd_ref, bd_ref,
             out_ref, psum, cnt):
        k = pl.program_id(0)

        @pl.when(k == 0)
        def _():
            psum[...] = jnp.zeros((G, 64), jnp.float32)
            cnt[...] = jnp.zeros((G, 16), jnp.float32)

        agg5 = (pa_ref[...] + pb_ref[...]) * dinv_ref[...]
        h5 = _l2relu(s5_ref[...] + agg5)
        m = (ib_ref[...] == lax.broadcasted_iota(jnp.int32, (RBLK, G), 1))
        m = m.astype(jnp.float32)
        cT = (((0,), (0,)), ((), ()))
        psum[...] += lax.dot_general(
            m, h5, cT, precision=lax.Precision.HIGHEST,
            preferred_element_type=jnp.float32)
        cnt[...] += lax.dot_general(
            m, jnp.ones((RBLK, 16), jnp.float32), cT,
            precision=lax.Precision.HIGHEST,
            preferred_element_type=jnp.float32)

        @pl.when(k == NB - 1)
        def _():
            pooled = psum[...] * (1.0 / jnp.maximum(cnt[:, 0:1], 1.0))
            out_ref[...] = jnp.tanh(_mm(pooled, wd_ref[...]) + bd_ref[0:1, :])

    return pl.pallas_call(
        body,
        grid=(NB,),
        in_specs=[_row_spec(64), pa, pb, _row_spec(64), _row_spec(64),
                  _full_spec((64, 16)), _full_spec((8, 16))],
        out_specs=_full_spec((G, 16)),
        out_shape=jax.ShapeDtypeStruct((G, 16), jnp.float32),
        scratch_shapes=[
            pltpu.VMEM((G, 64), jnp.float32),
            pltpu.VMEM((G, 16), jnp.float32),
        ],
    )(s5, part5, part5, dinv, ib, wd, bd)


def _pad_bias(b, d):
    return jnp.broadcast_to(b[None, :], (8, d))


def kernel(x, edge_index, i, W1, b1, W2, b2, W3, b3, W4, b4, W5, b5, Wd, bd):
    f32 = jnp.float32
    xp = jnp.pad(x, ((0, N_PAD - N), (0, 0)))
    src = jnp.concatenate(
        [edge_index[0], jnp.zeros((E_PAD - E,), jnp.int32)]).reshape(-1, CH)
    dst = jnp.concatenate(
        [edge_index[1], jnp.full((E_PAD - E,), N, jnp.int32)]).reshape(-1, CH)
    ip = jnp.concatenate([i, jnp.full((N_PAD - N,), G, jnp.int32)])
    ib = jnp.broadcast_to(ip[:, None], (N_PAD, G))
    wd16 = jnp.pad(Wd, ((0, 0), (0, 6)))
    bd16 = _pad_bias(jnp.pad(bd, (0, 6)), 16)

    ones16 = jnp.pad(jnp.ones((CH, 1), f32), ((0, 0), (0, DEG_W - 1)))
    zeros16 = jnp.zeros((RPT, DEG_W), f32)
    deg_part = _make_sc_deg()(dst, ones16, zeros16)

    s1, p1 = _tc1(xp, W1, _pad_bias(b1, 64))
    part1 = _sc_aggregate(p1, src, dst)
    h1, dinv = _tc2(s1, part1, deg_part)

    part2 = _sc_aggregate(h1, src, dst)
    s3, p3 = _tc3(h1, part2, dinv, W2, _pad_bias(b2, 256), W3, _pad_bias(b3, 32))

    part3 = _sc_aggregate(p3, src, dst)
    h3 = _tc4(s3, part3, dinv)

    part4 = _sc_aggregate(h3, src, dst)
    s5, p5 = _tc5(h3, part4, dinv, W4, _pad_bias(b4, 64), W5, _pad_bias(b5, 64))

    part5 = _sc_aggregate(p5, src, dst)
    out16 = _tc6(s5, part5, dinv, ib, wd16, bd16)
    return out16[:, :10].astype(f32)


# speedup vs baseline: 7.8910x; 1.1416x over previous
import functools

import jax
import jax.numpy as jnp
from jax import lax
from jax.experimental import pallas as pl
from jax.experimental.pallas import tpu as pltpu
from jax.experimental.pallas import tpu_sc as plsc

N = 10000
E = 320000
G = 64
N_PAD = 10240
E_PAD = 327680
NC = 2
NS = 16
CH = 128
EPT = E_PAD // (NC * NS)
NCHUNK = EPT // CH
RPT = N_PAD // NS
RBLK = 1024
NB = N_PAD // RBLK


_K_FOR_D = {64: 4, 32: 5}


@functools.lru_cache(maxsize=None)
def _make_sc_agg(d):
    K = _K_FOR_D[d]
    NGRP = NCHUNK // K
    NPAIR = (NGRP - 2) // 2
    stage_vals = d == 32
    mesh = plsc.VectorSubcoreMesh(
        core_axis_name="c", subcore_axis_name="s",
        num_cores=NC, num_subcores=NS)

    @functools.partial(
        pl.kernel,
        mesh=mesh,
        compiler_params=pltpu.CompilerParams(use_tc_tiling_on_sc=False),
        out_type=jax.ShapeDtypeStruct((NC * N_PAD, d), jnp.float32),
        scratch_types=(
            [pltpu.VMEM((NCHUNK, CH), jnp.int32)]
            + [pltpu.VMEM((NCHUNK, CH), jnp.int32)]
            + [pltpu.VMEM((CH, d), jnp.float32) for _ in range(2 * K)]
            + [pltpu.VMEM_SHARED((N_PAD, d), jnp.float32)]
            + ([pltpu.VMEM_SHARED((N_PAD, d), jnp.float32)]
               if stage_vals else [])
            + [pltpu.SemaphoreType.DMA for _ in range(4)]
        ),
    )
    def agg(vals, src2, dst2, zeros, out, *scr):
        src_all = scr[0]
        dst_all = scr[1]
        rows_a = scr[2:2 + K]
        rows_b = scr[2 + K:2 + 2 * K]
        acc = scr[2 + 2 * K]
        off = 3 + 2 * K
        if stage_vals:
            vals_sp = scr[off]
            off += 1
        else:
            vals_sp = vals
        gs_a, gs_b, ss_a, ss_b = scr[off:off + 4]
        c = lax.axis_index("c")
        s = lax.axis_index("s")
        pltpu.sync_copy(zeros, acc.at[pl.ds(s * RPT, RPT)])
        row0 = (c * NS + s) * NCHUNK
        pltpu.sync_copy(src2.at[pl.ds(row0, NCHUNK)], src_all)
        pltpu.sync_copy(dst2.at[pl.ds(row0, NCHUNK)], dst_all)
        if stage_vals:
            pltpu.sync_copy(
                vals.at[pl.ds(s * RPT, RPT)], vals_sp.at[pl.ds(s * RPT, RPT)])
        plsc.subcore_barrier()

        def fire_g(rows, gsem, g):
            for b in range(K):
                pltpu.async_copy(
                    vals_sp.at[src_all.at[g * K + b]], rows[b], gsem)

        def wait_g(rows, gsem, g):
            for b in range(K):
                pltpu.make_async_copy(
                    vals_sp.at[src_all.at[g * K + b]], rows[b], gsem).wait()

        def fire_s(rows, ssem, g):
            for b in range(K):
                pltpu.async_copy(
                    rows[b], acc.at[dst_all.at[g * K + b]], ssem, add=True)

        def drain_s(rows, ssem, g):
            for b in range(K):
                pltpu.make_async_copy(
                    rows[b], acc.at[dst_all.at[g * K + b]], ssem).wait()

        fire_g(rows_a, gs_a, 0)
        wait_g(rows_a, gs_a, 0)
        fire_g(rows_b, gs_b, 1)
        fire_s(rows_a, ss_a, 0)

        def pair(j, carry):
            gb = 2 * j + 1
            ga = 2 * j + 2
            wait_g(rows_b, gs_b, gb)
            drain_s(rows_a, ss_a, ga - 2)
            fire_g(rows_a, gs_a, ga)
            fire_s(rows_b, ss_b, gb)
            wait_g(rows_a, gs_a, ga)
            drain_s(rows_b, ss_b, gb)
            fire_g(rows_b, gs_b, ga + 1)
            fire_s(rows_a, ss_a, ga)
            return carry

        lax.fori_loop(0, NPAIR, pair, 0)
        wait_g(rows_b, gs_b, NGRP - 1)
        drain_s(rows_a, ss_a, NGRP - 2)
        fire_s(rows_b, ss_b, NGRP - 1)
        drain_s(rows_b, ss_b, NGRP - 1)

        plsc.subcore_barrier()
        pltpu.sync_copy(
            acc.at[pl.ds(s * RPT, RPT)],
            out.at[pl.ds(c * N_PAD + s * RPT, RPT)],
        )

    return agg


def _sc_aggregate(vals, src, dst):
    d = vals.shape[1]
    zeros = jnp.zeros((RPT, d), jnp.float32)
    return _make_sc_agg(d)(vals, src, dst, zeros)


DEG_W = 16
DEG_K = 8


def _make_sc_deg():
    mesh = plsc.VectorSubcoreMesh(
        core_axis_name="c", subcore_axis_name="s",
        num_cores=NC, num_subcores=NS)

    @functools.partial(
        pl.kernel,
        mesh=mesh,
        compiler_params=pltpu.CompilerParams(use_tc_tiling_on_sc=False),
        out_type=jax.ShapeDtypeStruct((NC * N_PAD, DEG_W), jnp.float32),
        scratch_types=(
            [pltpu.VMEM((NCHUNK, CH), jnp.int32)]
            + [pltpu.VMEM((CH, DEG_W), jnp.float32)]
            + [pltpu.VMEM_SHARED((N_PAD, DEG_W), jnp.float32)]
            + [pltpu.SemaphoreType.DMA]
        ),
    )
    def deg(dst2, ones, zeros, out, dst_all, ones_v, acc, ssem):
        c = lax.axis_index("c")
        s = lax.axis_index("s")
        pltpu.sync_copy(zeros, acc.at[pl.ds(s * RPT, RPT)])
        row0 = (c * NS + s) * NCHUNK
        pltpu.sync_copy(dst2.at[pl.ds(row0, NCHUNK)], dst_all)
        pltpu.sync_copy(ones, ones_v)
        plsc.subcore_barrier()

        def group(j, carry):
            g0 = j * DEG_K
            for b in range(DEG_K):
                pltpu.async_copy(
                    ones_v, acc.at[dst_all.at[g0 + b]], ssem, add=True)
            for b in range(DEG_K):
                pltpu.make_async_copy(
                    ones_v, acc.at[dst_all.at[g0 + b]], ssem).wait()
            return carry

        lax.fori_loop(0, NCHUNK // DEG_K, group, 0)
        plsc.subcore_barrier()
        pltpu.sync_copy(
            acc.at[pl.ds(s * RPT, RPT)],
            out.at[pl.ds(c * N_PAD + s * RPT, RPT)],
        )

    return deg


def _mm(a, b):
    return lax.dot_general(
        a, b, (((1,), (0,)), ((), ())),
        precision=lax.Precision.HIGHEST,
        preferred_element_type=jnp.float32,
    )


def _l2relu(t):
    nrm = lax.rsqrt(jnp.maximum(jnp.sum(t * t, -1, keepdims=True), 1e-12))
    return jnp.maximum(t * nrm, 0.0)


def _row_spec(d):
    return pl.BlockSpec((RBLK, d), lambda k: (k, 0))


def _part_specs(d):
    return (
        pl.BlockSpec((RBLK, d), lambda k: (k, 0)),
        pl.BlockSpec((RBLK, d), lambda k: (k + NB, 0)),
    )


def _full_spec(shape):
    nd = len(shape)
    return pl.BlockSpec(shape, lambda k: (0,) * nd)


def _tc1_body(x_ref, w_ref, b_ref, s1_ref, p1_ref):
    x = x_ref[...]
    w = w_ref[...]
    s1_ref[...] = _mm(x, w[:128]) + b_ref[0:1, :]
    p1_ref[...] = _mm(x, w[128:])


def _tc1(x, w1, b1):
    return pl.pallas_call(
        _tc1_body,
        grid=(NB,),
        in_specs=[_row_spec(128), _full_spec((256, 64)), _full_spec((8, 64))],
        out_specs=[_row_spec(64), _row_spec(64)],
        out_shape=[
            jax.ShapeDtypeStruct((N_PAD, 64), jnp.float32),
            jax.ShapeDtypeStruct((N_PAD, 64), jnp.float32),
        ],
    )(x, w1, b1)


def _tc2_body(s1_ref, pa_ref, pb_ref, da_ref, db_ref, h1_ref, dinv_ref):
    p = pa_ref[...] + pb_ref[...]
    deg = jnp.sum(da_ref[...] + db_ref[...], -1, keepdims=True)
    dinv = 1.0 / jnp.maximum(deg, 1.0)
    t = s1_ref[...] + p * dinv
    h1_ref[...] = _l2relu(t)
    dinv_ref[...] = jnp.broadcast_to(dinv, (RBLK, 64))


def _tc2(s1, part1, deg_part):
    pa, pb = _part_specs(64)
    da, db = _part_specs(DEG_W)
    return pl.pallas_call(
        _tc2_body,
        grid=(NB,),
        in_specs=[_row_spec(64), pa, pb, da, db],
        out_specs=[_row_spec(64), _row_spec(64)],
        out_shape=[
            jax.ShapeDtypeStruct((N_PAD, 64), jnp.float32),
            jax.ShapeDtypeStruct((N_PAD, 64), jnp.float32),
        ],
    )(s1, part1, part1, deg_part, deg_part)


def _tc3(h1, part2, dinv, w2, b2, w3, b3):
    pa, pb = _part_specs(64)

    def body(h1_ref, pa_ref, pb_ref, dinv_ref, w2_ref, b2_ref, w3_ref, b3_ref,
             s3_ref, p3_ref):
        h1 = h1_ref[...]
        agg2 = (pa_ref[...] + pb_ref[...]) * dinv_ref[...]
        w2 = w2_ref[...]
        h2 = _l2relu(_mm(h1, w2[:64]) + _mm(agg2, w2[64:]) + b2_ref[0:1, :])
        w3 = w3_ref[...]
        s3_ref[...] = _mm(h2, w3[:256]) + b3_ref[0:1, :]
        p3_ref[...] = _mm(h2, w3[256:])

    return pl.pallas_call(
        body,
        grid=(NB,),
        in_specs=[_row_spec(64), pa, pb, _row_spec(64),
                  _full_spec((128, 256)), _full_spec((8, 256)),
                  _full_spec((512, 32)), _full_spec((8, 32))],
        out_specs=[_row_spec(32), _row_spec(32)],
        out_shape=[
            jax.ShapeDtypeStruct((N_PAD, 32), jnp.float32),
            jax.ShapeDtypeStruct((N_PAD, 32), jnp.float32),
        ],
    )(h1, part2, part2, dinv, w2, b2, w3, b3)


def _tc4(s3, part3, dinv):
    pa, pb = _part_specs(32)

    def body(s3_ref, pa_ref, pb_ref, dinv_ref, h3_ref):
        agg3 = (pa_ref[...] + pb_ref[...]) * dinv_ref[:, :32]
        h3_ref[...] = _l2relu(s3_ref[...] + agg3)

    return pl.pallas_call(
        body,
        grid=(NB,),
        in_specs=[_row_spec(32), pa, pb, _row_spec(64)],
        out_specs=_row_spec(32),
        out_shape=jax.ShapeDtypeStruct((N_PAD, 32), jnp.float32),
    )(s3, part3, part3, dinv)


def _tc5(h3, part4, dinv, w4, b4, w5, b5):
    pa, pb = _part_specs(32)

    def body(h3_ref, pa_ref, pb_ref, dinv_ref, w4_ref, b4_ref, w5_ref, b5_ref,
             s5_ref, p5_ref):
        h3 = h3_ref[...]
        agg4 = (pa_ref[...] + pb_ref[...]) * dinv_ref[:, :32]
        w4 = w4_ref[...]
        h4 = _l2relu(_mm(h3, w4[:32]) + _mm(agg4, w4[32:]) + b4_ref[0:1, :])
        w5 = w5_ref[...]
        s5_ref[...] = _mm(h4, w5[:64]) + b5_ref[0:1, :]
        p5_ref[...] = _mm(h4, w5[64:])

    return pl.pallas_call(
        body,
        grid=(NB,),
        in_specs=[_row_spec(32), pa, pb, _row_spec(64),
                  _full_spec((64, 64)), _full_spec((8, 64)),
                  _full_spec((128, 64)), _full_spec((8, 64))],
        out_specs=[_row_spec(64), _row_spec(64)],
        out_shape=[
            jax.ShapeDtypeStruct((N_PAD, 64), jnp.float32),
            jax.ShapeDtypeStruct((N_PAD, 64), jnp.float32),
        ],
    )(h3, part4, part4, dinv, w4, b4, w5, b5)


def _tc6(s5, part5, dinv, ib, wd, bd):
    pa, pb = _part_specs(64)

    def body(s5_ref, pa_ref, pb_ref, dinv_ref, ib_ref, wd_ref, bd_ref,
             out_ref, psum, cnt):
        k = pl.program_id(0)

        @pl.when(k == 0)
        def _():
            psum[...] = jnp.zeros((G, 64), jnp.float32)
            cnt[...] = jnp.zeros((G, 16), jnp.float32)

        agg5 = (pa_ref[...] + pb_ref[...]) * dinv_ref[...]
        h5 = _l2relu(s5_ref[...] + agg5)
        m = (ib_ref[...] == lax.broadcasted_iota(jnp.int32, (RBLK, G), 1))
        m = m.astype(jnp.float32)
        cT = (((0,), (0,)), ((), ()))
        psum[...] += lax.dot_general(
            m, h5, cT, precision=lax.Precision.HIGHEST,
            preferred_element_type=jnp.float32)
        cnt[...] += lax.dot_general(
            m, jnp.ones((RBLK, 16), jnp.float32), cT,
            precision=lax.Precision.HIGHEST,
            preferred_element_type=jnp.float32)

        @pl.when(k == NB - 1)
        def _():
            pooled = psum[...] * (1.0 / jnp.maximum(cnt[:, 0:1], 1.0))
            out_ref[...] = jnp.tanh(_mm(pooled, wd_ref[...]) + bd_ref[0:1, :])

    return pl.pallas_call(
        body,
        grid=(NB,),
        in_specs=[_row_spec(64), pa, pb, _row_spec(64), _row_spec(64),
                  _full_spec((64, 16)), _full_spec((8, 16))],
        out_specs=_full_spec((G, 16)),
        out_shape=jax.ShapeDtypeStruct((G, 16), jnp.float32),
        scratch_shapes=[
            pltpu.VMEM((G, 64), jnp.float32),
            pltpu.VMEM((G, 16), jnp.float32),
        ],
    )(s5, part5, part5, dinv, ib, wd, bd)


def _pad_bias(b, d):
    return jnp.broadcast_to(b[None, :], (8, d))


def kernel(x, edge_index, i, W1, b1, W2, b2, W3, b3, W4, b4, W5, b5, Wd, bd):
    f32 = jnp.float32
    xp = jnp.pad(x, ((0, N_PAD - N), (0, 0)))
    src = jnp.concatenate(
        [edge_index[0], jnp.zeros((E_PAD - E,), jnp.int32)]).reshape(-1, CH)
    dst = jnp.concatenate(
        [edge_index[1], jnp.full((E_PAD - E,), N, jnp.int32)]).reshape(-1, CH)
    ip = jnp.concatenate([i, jnp.full((N_PAD - N,), G, jnp.int32)])
    ib = jnp.broadcast_to(ip[:, None], (N_PAD, G))
    wd16 = jnp.pad(Wd, ((0, 0), (0, 6)))
    bd16 = _pad_bias(jnp.pad(bd, (0, 6)), 16)

    ones16 = jnp.pad(jnp.ones((CH, 1), f32), ((0, 0), (0, DEG_W - 1)))
    zeros16 = jnp.zeros((RPT, DEG_W), f32)
    deg_part = _make_sc_deg()(dst, ones16, zeros16)

    s1, p1 = _tc1(xp, W1, _pad_bias(b1, 64))
    part1 = _sc_aggregate(p1, src, dst)
    h1, dinv = _tc2(s1, part1, deg_part)

    part2 = _sc_aggregate(h1, src, dst)
    s3, p3 = _tc3(h1, part2, dinv, W2, _pad_bias(b2, 256), W3, _pad_bias(b3, 32))

    part3 = _sc_aggregate(p3, src, dst)
    h3 = _tc4(s3, part3, dinv)

    part4 = _sc_aggregate(h3, src, dst)
    s5, p5 = _tc5(h3, part4, dinv, W4, _pad_bias(b4, 64), W5, _pad_bias(b5, 64))

    part5 = _sc_aggregate(p5, src, dst)
    out16 = _tc6(s5, part5, dinv, ib, wd16, bd16)
    return out16[:, :10].astype(f32)


# speedup vs baseline: 12.5853x; 1.5949x over previous
import functools

import jax
import jax.numpy as jnp
from jax import lax
from jax.experimental import pallas as pl
from jax.experimental.pallas import tpu as pltpu
from jax.experimental.pallas import tpu_sc as plsc

N = 10000
E = 320000
G = 64
N_PAD = 10240
E_PAD = 327680
NC = 2
NS = 16
CH = 128
EPT = E_PAD // (NC * NS)
NCHUNK = EPT // CH
RPT = N_PAD // NS
RBLK = 1024
NB = N_PAD // RBLK


_K_FOR_D = {64: 4, 32: 5}


@functools.lru_cache(maxsize=None)
def _make_sc_agg(d):
    K = _K_FOR_D[d]
    NGRP = NCHUNK // K
    NPAIR = (NGRP - 2) // 2
    stage_vals = d == 32
    mesh = plsc.VectorSubcoreMesh(
        core_axis_name="c", subcore_axis_name="s",
        num_cores=NC, num_subcores=NS)

    @functools.partial(
        pl.kernel,
        mesh=mesh,
        compiler_params=pltpu.CompilerParams(use_tc_tiling_on_sc=False),
        out_type=jax.ShapeDtypeStruct((NC * N_PAD, d), jnp.float32),
        scratch_types=(
            [pltpu.VMEM((NCHUNK, CH), jnp.int32)]
            + [pltpu.VMEM((NCHUNK, CH), jnp.int32)]
            + [pltpu.VMEM((CH, d), jnp.float32) for _ in range(2 * K)]
            + [pltpu.VMEM_SHARED((N_PAD, d), jnp.float32)]
            + ([pltpu.VMEM_SHARED((N_PAD, d), jnp.float32)]
               if stage_vals else [])
            + [pltpu.SemaphoreType.DMA for _ in range(4)]
        ),
    )
    def agg(vals, src2, dst2, zeros, out, *scr):
        src_all = scr[0]
        dst_all = scr[1]
        rows_a = scr[2:2 + K]
        rows_b = scr[2 + K:2 + 2 * K]
        acc = scr[2 + 2 * K]
        off = 3 + 2 * K
        if stage_vals:
            vals_sp = scr[off]
            off += 1
        else:
            vals_sp = vals
        gs_a, gs_b, ss_a, ss_b = scr[off:off + 4]
        c = lax.axis_index("c")
        s = lax.axis_index("s")
        pltpu.sync_copy(zeros, acc.at[pl.ds(s * RPT, RPT)])
        row0 = (c * NS + s) * NCHUNK
        pltpu.sync_copy(src2.at[pl.ds(row0, NCHUNK)], src_all)
        pltpu.sync_copy(dst2.at[pl.ds(row0, NCHUNK)], dst_all)
        if stage_vals:
            pltpu.sync_copy(
                vals.at[pl.ds(s * RPT, RPT)], vals_sp.at[pl.ds(s * RPT, RPT)])
        plsc.subcore_barrier()

        def fire_g(rows, gsem, g):
            for b in range(K):
                pltpu.async_copy(
                    vals_sp.at[src_all.at[g * K + b]], rows[b], gsem)

        def wait_g(rows, gsem, g):
            for b in range(K):
                pltpu.make_async_copy(
                    vals_sp.at[src_all.at[g * K + b]], rows[b], gsem).wait()

        def fire_s(rows, ssem, g):
            for b in range(K):
                pltpu.async_copy(
                    rows[b], acc.at[dst_all.at[g * K + b]], ssem, add=True)

        def drain_s(rows, ssem, g):
            for b in range(K):
                pltpu.make_async_copy(
                    rows[b], acc.at[dst_all.at[g * K + b]], ssem).wait()

        fire_g(rows_a, gs_a, 0)
        wait_g(rows_a, gs_a, 0)
        fire_g(rows_b, gs_b, 1)
        fire_s(rows_a, ss_a, 0)

        def pair(j, carry):
            gb = 2 * j + 1
            ga = 2 * j + 2
            wait_g(rows_b, gs_b, gb)
            drain_s(rows_a, ss_a, ga - 2)
            fire_g(rows_a, gs_a, ga)
            fire_s(rows_b, ss_b, gb)
            wait_g(rows_a, gs_a, ga)
            drain_s(rows_b, ss_b, gb)
            fire_g(rows_b, gs_b, ga + 1)
            fire_s(rows_a, ss_a, ga)
            return carry

        lax.fori_loop(0, NPAIR, pair, 0)
        wait_g(rows_b, gs_b, NGRP - 1)
        drain_s(rows_a, ss_a, NGRP - 2)
        fire_s(rows_b, ss_b, NGRP - 1)
        drain_s(rows_b, ss_b, NGRP - 1)

        plsc.subcore_barrier()
        pltpu.sync_copy(
            acc.at[pl.ds(s * RPT, RPT)],
            out.at[pl.ds(c * N_PAD + s * RPT, RPT)],
        )

    return agg


def _sc_aggregate(vals, src, dst):
    d = vals.shape[1]
    zeros = jnp.zeros((RPT, d), jnp.float32)
    return _make_sc_agg(d)(vals, src, dst, zeros)


def _sc_aggregate64(vals, src, dst):
    lo = _sc_aggregate(vals[:, :32], src, dst)
    hi = _sc_aggregate(vals[:, 32:], src, dst)
    return lo, hi


DEG_W = 16
DEG_K = 8


def _make_sc_deg():
    mesh = plsc.VectorSubcoreMesh(
        core_axis_name="c", subcore_axis_name="s",
        num_cores=NC, num_subcores=NS)

    @functools.partial(
        pl.kernel,
        mesh=mesh,
        compiler_params=pltpu.CompilerParams(use_tc_tiling_on_sc=False),
        out_type=jax.ShapeDtypeStruct((NC * N_PAD, DEG_W), jnp.float32),
        scratch_types=(
            [pltpu.VMEM((NCHUNK, CH), jnp.int32)]
            + [pltpu.VMEM((CH, DEG_W), jnp.float32)]
            + [pltpu.VMEM_SHARED((N_PAD, DEG_W), jnp.float32)]
            + [pltpu.SemaphoreType.DMA]
        ),
    )
    def deg(dst2, ones, zeros, out, dst_all, ones_v, acc, ssem):
        c = lax.axis_index("c")
        s = lax.axis_index("s")
        pltpu.sync_copy(zeros, acc.at[pl.ds(s * RPT, RPT)])
        row0 = (c * NS + s) * NCHUNK
        pltpu.sync_copy(dst2.at[pl.ds(row0, NCHUNK)], dst_all)
        pltpu.sync_copy(ones, ones_v)
        plsc.subcore_barrier()

        def group(j, carry):
            g0 = j * DEG_K
            for b in range(DEG_K):
                pltpu.async_copy(
                    ones_v, acc.at[dst_all.at[g0 + b]], ssem, add=True)
            for b in range(DEG_K):
                pltpu.make_async_copy(
                    ones_v, acc.at[dst_all.at[g0 + b]], ssem).wait()
            return carry

        lax.fori_loop(0, NCHUNK // DEG_K, group, 0)
        plsc.subcore_barrier()
        pltpu.sync_copy(
            acc.at[pl.ds(s * RPT, RPT)],
            out.at[pl.ds(c * N_PAD + s * RPT, RPT)],
        )

    return deg


def _mm(a, b):
    return lax.dot_general(
        a, b, (((1,), (0,)), ((), ())),
        precision=lax.Precision.HIGHEST,
        preferred_element_type=jnp.float32,
    )


def _l2relu(t):
    nrm = lax.rsqrt(jnp.maximum(jnp.sum(t * t, -1, keepdims=True), 1e-12))
    return jnp.maximum(t * nrm, 0.0)


def _row_spec(d):
    return pl.BlockSpec((RBLK, d), lambda k: (k, 0))


def _part_specs(d):
    return (
        pl.BlockSpec((RBLK, d), lambda k: (k, 0)),
        pl.BlockSpec((RBLK, d), lambda k: (k + NB, 0)),
    )


def _full_spec(shape):
    nd = len(shape)
    return pl.BlockSpec(shape, lambda k: (0,) * nd)


def _tc1_body(x_ref, w_ref, b_ref, s1_ref, p1_ref):
    x = x_ref[...]
    w = w_ref[...]
    s1_ref[...] = _mm(x, w[:128]) + b_ref[0:1, :]
    p1_ref[...] = _mm(x, w[128:])


def _tc1(x, w1, b1):
    return pl.pallas_call(
        _tc1_body,
        grid=(NB,),
        in_specs=[_row_spec(128), _full_spec((256, 64)), _full_spec((8, 64))],
        out_specs=[_row_spec(64), _row_spec(64)],
        out_shape=[
            jax.ShapeDtypeStruct((N_PAD, 64), jnp.float32),
            jax.ShapeDtypeStruct((N_PAD, 64), jnp.float32),
        ],
    )(x, w1, b1)


def _sum64(la_ref, lb_ref, ha_ref, hb_ref):
    return jnp.concatenate(
        [la_ref[...] + lb_ref[...], ha_ref[...] + hb_ref[...]], axis=1)


def _part_specs64():
    pa, pb = _part_specs(32)
    return (pa, pb, pa, pb)


def _tc2_body(s1_ref, la_ref, lb_ref, ha_ref, hb_ref, da_ref, db_ref,
              h1_ref, dinv_ref):
    p = _sum64(la_ref, lb_ref, ha_ref, hb_ref)
    deg = jnp.sum(da_ref[...] + db_ref[...], -1, keepdims=True)
    dinv = 1.0 / jnp.maximum(deg, 1.0)
    t = s1_ref[...] + p * dinv
    h1_ref[...] = _l2relu(t)
    dinv_ref[...] = jnp.broadcast_to(dinv, (RBLK, 64))


def _tc2(s1, part1, deg_part):
    la, lb, ha, hb = _part_specs64()
    da, db = _part_specs(DEG_W)
    return pl.pallas_call(
        _tc2_body,
        grid=(NB,),
        in_specs=[_row_spec(64), la, lb, ha, hb, da, db],
        out_specs=[_row_spec(64), _row_spec(64)],
        out_shape=[
            jax.ShapeDtypeStruct((N_PAD, 64), jnp.float32),
            jax.ShapeDtypeStruct((N_PAD, 64), jnp.float32),
        ],
    )(s1, part1[0], part1[0], part1[1], part1[1], deg_part, deg_part)


def _tc3(h1, part2, dinv, w2, b2, w3, b3):
    la, lb, ha, hb = _part_specs64()

    def body(h1_ref, la_ref, lb_ref, ha_ref, hb_ref, dinv_ref, w2_ref,
             b2_ref, w3_ref, b3_ref, s3_ref, p3_ref):
        h1 = h1_ref[...]
        agg2 = _sum64(la_ref, lb_ref, ha_ref, hb_ref) * dinv_ref[...]
        w2 = w2_ref[...]
        h2 = _l2relu(_mm(h1, w2[:64]) + _mm(agg2, w2[64:]) + b2_ref[0:1, :])
        w3 = w3_ref[...]
        s3_ref[...] = _mm(h2, w3[:256]) + b3_ref[0:1, :]
        p3_ref[...] = _mm(h2, w3[256:])

    return pl.pallas_call(
        body,
        grid=(NB,),
        in_specs=[_row_spec(64), la, lb, ha, hb, _row_spec(64),
                  _full_spec((128, 256)), _full_spec((8, 256)),
                  _full_spec((512, 32)), _full_spec((8, 32))],
        out_specs=[_row_spec(32), _row_spec(32)],
        out_shape=[
            jax.ShapeDtypeStruct((N_PAD, 32), jnp.float32),
            jax.ShapeDtypeStruct((N_PAD, 32), jnp.float32),
        ],
    )(h1, part2[0], part2[0], part2[1], part2[1], dinv, w2, b2, w3, b3)


def _tc4(s3, part3, dinv):
    pa, pb = _part_specs(32)

    def body(s3_ref, pa_ref, pb_ref, dinv_ref, h3_ref):
        agg3 = (pa_ref[...] + pb_ref[...]) * dinv_ref[:, :32]
        h3_ref[...] = _l2relu(s3_ref[...] + agg3)

    return pl.pallas_call(
        body,
        grid=(NB,),
        in_specs=[_row_spec(32), pa, pb, _row_spec(64)],
        out_specs=_row_spec(32),
        out_shape=jax.ShapeDtypeStruct((N_PAD, 32), jnp.float32),
    )(s3, part3, part3, dinv)


def _tc5(h3, part4, dinv, w4, b4, w5, b5):
    pa, pb = _part_specs(32)

    def body(h3_ref, pa_ref, pb_ref, dinv_ref, w4_ref, b4_ref, w5_ref, b5_ref,
             s5_ref, p5_ref):
        h3 = h3_ref[...]
        agg4 = (pa_ref[...] + pb_ref[...]) * dinv_ref[:, :32]
        w4 = w4_ref[...]
        h4 = _l2relu(_mm(h3, w4[:32]) + _mm(agg4, w4[32:]) + b4_ref[0:1, :])
        w5 = w5_ref[...]
        s5_ref[...] = _mm(h4, w5[:64]) + b5_ref[0:1, :]
        p5_ref[...] = _mm(h4, w5[64:])

    return pl.pallas_call(
        body,
        grid=(NB,),
        in_specs=[_row_spec(32), pa, pb, _row_spec(64),
                  _full_spec((64, 64)), _full_spec((8, 64)),
                  _full_spec((128, 64)), _full_spec((8, 64))],
        out_specs=[_row_spec(64), _row_spec(64)],
        out_shape=[
            jax.ShapeDtypeStruct((N_PAD, 64), jnp.float32),
            jax.ShapeDtypeStruct((N_PAD, 64), jnp.float32),
        ],
    )(h3, part4, part4, dinv, w4, b4, w5, b5)


def _tc6(s5, part5, dinv, ib, wd, bd):
    la, lb, ha, hb = _part_specs64()

    def body(s5_ref, la_ref, lb_ref, ha_ref, hb_ref, dinv_ref, ib_ref,
             wd_ref, bd_ref, out_ref, psum, cnt):
        k = pl.program_id(0)

        @pl.when(k == 0)
        def _():
            psum[...] = jnp.zeros((G, 64), jnp.float32)
            cnt[...] = jnp.zeros((G, 16), jnp.float32)

        agg5 = _sum64(la_ref, lb_ref, ha_ref, hb_ref) * dinv_ref[...]
        h5 = _l2relu(s5_ref[...] + agg5)
        m = (ib_ref[...] == lax.broadcasted_iota(jnp.int32, (RBLK, G), 1))
        m = m.astype(jnp.float32)
        cT = (((0,), (0,)), ((), ()))
        psum[...] += lax.dot_general(
            m, h5, cT, precision=lax.Precision.HIGHEST,
            preferred_element_type=jnp.float32)
        cnt[...] += lax.dot_general(
            m, jnp.ones((RBLK, 16), jnp.float32), cT,
            precision=lax.Precision.HIGHEST,
            preferred_element_type=jnp.float32)

        @pl.when(k == NB - 1)
        def _():
            pooled = psum[...] * (1.0 / jnp.maximum(cnt[:, 0:1], 1.0))
            out_ref[...] = jnp.tanh(_mm(pooled, wd_ref[...]) + bd_ref[0:1, :])

    return pl.pallas_call(
        body,
        grid=(NB,),
        in_specs=[_row_spec(64), la, lb, ha, hb, _row_spec(64), _row_spec(64),
                  _full_spec((64, 16)), _full_spec((8, 16))],
        out_specs=_full_spec((G, 16)),
        out_shape=jax.ShapeDtypeStruct((G, 16), jnp.float32),
        scratch_shapes=[
            pltpu.VMEM((G, 64), jnp.float32),
            pltpu.VMEM((G, 16), jnp.float32),
        ],
    )(s5, part5[0], part5[0], part5[1], part5[1], dinv, ib, wd, bd)


def _pad_bias(b, d):
    return jnp.broadcast_to(b[None, :], (8, d))


def kernel(x, edge_index, i, W1, b1, W2, b2, W3, b3, W4, b4, W5, b5, Wd, bd):
    f32 = jnp.float32
    xp = jnp.pad(x, ((0, N_PAD - N), (0, 0)))
    src = jnp.concatenate(
        [edge_index[0], jnp.zeros((E_PAD - E,), jnp.int32)]).reshape(-1, CH)
    dst = jnp.concatenate(
        [edge_index[1], jnp.full((E_PAD - E,), N, jnp.int32)]).reshape(-1, CH)
    ip = jnp.concatenate([i, jnp.full((N_PAD - N,), G, jnp.int32)])
    ib = jnp.broadcast_to(ip[:, None], (N_PAD, G))
    wd16 = jnp.pad(Wd, ((0, 0), (0, 6)))
    bd16 = _pad_bias(jnp.pad(bd, (0, 6)), 16)

    ones16 = jnp.pad(jnp.ones((CH, 1), f32), ((0, 0), (0, DEG_W - 1)))
    zeros16 = jnp.zeros((RPT, DEG_W), f32)
    deg_part = _make_sc_deg()(dst, ones16, zeros16)

    s1, p1 = _tc1(xp, W1, _pad_bias(b1, 64))
    part1 = _sc_aggregate64(p1, src, dst)
    h1, dinv = _tc2(s1, part1, deg_part)

    part2 = _sc_aggregate64(h1, src, dst)
    s3, p3 = _tc3(h1, part2, dinv, W2, _pad_bias(b2, 256), W3, _pad_bias(b3, 32))

    part3 = _sc_aggregate(p3, src, dst)
    h3 = _tc4(s3, part3, dinv)

    part4 = _sc_aggregate(h3, src, dst)
    s5, p5 = _tc5(h3, part4, dinv, W4, _pad_bias(b4, 64), W5, _pad_bias(b5, 64))

    part5 = _sc_aggregate64(p5, src, dst)
    out16 = _tc6(s5, part5, dinv, ib, wd16, bd16)
    return out16[:, :10].astype(f32)
